# baseline (speedup 1.0000x reference)
import jax
import jax.numpy as jnp
from jax.experimental import pallas as pl

TEMPERATURE = 1.0
NUM_STRATA = 10
MAX_SCALES = 4


def _p_kernel(e_ref, s_ref, p_ref):
    beta = TEMPERATURE * (1.0 + 0.1 * s_ref[...].astype(jnp.float32) / MAX_SCALES)
    e = jnp.clip(e_ref[...], -10.0, 10.0)
    p = jnp.exp(beta * e)
    p = jnp.clip(p, 1e-8)
    p_ref[...] = jnp.where(jnp.isfinite(p), p, jnp.full_like(p, 1e-8))


def kernel(entropy_map, node_scales, budget):
    N = entropy_map.shape[0]
    e2 = entropy_map.reshape(1000, 1000)
    s2 = node_scales.reshape(1000, 1000)
    p = pl.pallas_call(
        _p_kernel,
        out_shape=jax.ShapeDtypeStruct((1000, 1000), jnp.float32),
    )(e2, s2).reshape(N)

    prob_sum = jnp.sum(p)
    probs = jnp.where(prob_sum <= 1e-8, jnp.ones_like(p) / N, p / prob_sum)
    q = jnp.quantile(entropy_map, jnp.linspace(0.0, 1.0, NUM_STRATA + 1))
    stratum = jnp.clip(
        jnp.searchsorted(q[1:NUM_STRATA], entropy_map, side='right'), 0, NUM_STRATA - 1)
    counts = jnp.bincount(stratum, length=NUM_STRATA)
    u = jax.random.uniform(jax.random.key(42), (N,), minval=1e-12, maxval=1.0)
    gumbel = -jnp.log(-jnp.log(u))
    score = jnp.log(jnp.clip(probs, 1e-8)) + gumbel
    g = jnp.gcd(budget, N)
    b = jnp.maximum(1, (counts * (budget // g)) // (N // g))
    b = jnp.minimum(b, counts)
    order = jnp.argsort(-score)
    stratum_sorted = stratum[order]
    select_sorted = jnp.zeros((N,), dtype=jnp.bool_)
    for i in range(NUM_STRATA):
        in_i = stratum_sorted == i
        rank_i = jnp.cumsum(in_i.astype(jnp.int32)) - 1
        select_sorted = select_sorted | (in_i & (rank_i < b[i]))
    mask = jnp.zeros((N,), dtype=jnp.bool_).at[order].set(select_sorted)
    return probs, mask


# speedup vs baseline: 17.1486x; 17.1486x over previous
import functools

import numpy as np
import jax
import jax.numpy as jnp
from jax import lax
from jax.experimental import pallas as pl
from jax.experimental.pallas import tpu as pltpu
from jax.experimental.pallas import tpu_sc as plsc

N = 1_000_000
PAD = 1_000_448
ROWS = PAD // 128
NT = 32
NVT = 1954
BLOCKS = (512, 512, 512, 418)
NUM_STRATA = 10

_i32 = jnp.int32
_u32 = jnp.uint32
_f32 = jnp.float32


_QQ = np.array([0, 1203982323, 1212370931, 1217559543, 1220759539,
                1223959536, 1225948151, 1227548149, 1229148147, 1230748146,
                1232348144], dtype=np.uint32).view(np.float32)
_low9 = np.floor(_QQ).astype(np.int32)[1:10]
_high9 = np.ceil(_QQ).astype(np.int32)[1:10]
_hw9 = (_QQ - np.floor(_QQ)).astype(np.float32)[1:10]
_lw9 = (np.float32(1.0) - _hw9).astype(np.float32)
_RANKS18 = np.empty(18, np.int32)
_RANKS18[0::2] = _low9
_RANKS18[1::2] = _high9
_ranks_np = np.zeros((1, 128), np.int32)
_ranks_np[0, :18] = _RANKS18
_wts_np = np.zeros((1, 128), np.float32)
_wts_np[0, :9] = _lw9
_wts_np[0, 32:41] = _hw9

_CONSTS = {}


def _consts():
    if _CONSTS:
        return _CONSTS
    u = jax.random.uniform(jax.random.key(42), (N,), minval=1e-12, maxval=1.0)
    g = jnp.pad(-jnp.log(-jnp.log(u)), (0, PAD - N)).reshape(ROWS, 128)
    out = {"gumbel2d": g,
           "ranks_op": jnp.asarray(_ranks_np),
           "wts_op": jnp.asarray(_wts_np)}
    if not isinstance(g, jax.core.Tracer):
        _CONSTS.update(out)
    return out

_LOG1EM8 = np.float32(np.log(np.float32(1e-8)))
_SENT = 0x1FF


def _mono_u32(x):
    b = lax.bitcast_convert_type(x, _u32)
    neg = (b & np.uint32(0x80000000)) != 0
    return jnp.where(neg, ~b, b | np.uint32(0x80000000))


def _valid2d():
    r = lax.broadcasted_iota(_i32, (ROWS, 128), 0)
    c = lax.broadcasted_iota(_i32, (ROWS, 128), 1)
    return (r * 128 + c) < N


def _tc_ekey_body(e_ref, s_ref, ekey_ref, ppart_ref):
    e = e_ref[...]
    s = s_ref[...]
    ekey_ref[...] = _mono_u32(e)
    beta = np.float32(1.0) + np.float32(0.025) * s.astype(_f32)
    p = jnp.exp(beta * jnp.clip(e, -10.0, 10.0))
    p = jnp.where(_valid2d(), p, np.float32(0.0))
    ppart_ref[...] = jnp.sum(p, axis=0, keepdims=True)


def _tc_ekey(e2, s2):
    return pl.pallas_call(
        _tc_ekey_body,
        out_shape=[jax.ShapeDtypeStruct((ROWS, 128), _u32),
                   jax.ShapeDtypeStruct((1, 128), _f32)],
    )(e2, s2)


def _cumsum_lanes(x):
    col = lax.broadcasted_iota(_i32, x.shape, 1)
    for k in (1, 2, 4, 8, 16, 32, 64):
        sh = pltpu.roll(x, k, axis=1)
        x = x + jnp.where(col >= k, sh, jnp.zeros_like(x))
    return x


def _cumsum_rows(x):
    row = lax.broadcasted_iota(_i32, x.shape, 0)
    k = 1
    while k < x.shape[0]:
        sh = pltpu.roll(x, k, axis=0)
        x = x + jnp.where(row >= k, sh, jnp.zeros_like(x))
        k *= 2
    return x


_COL128 = None


def _col128():
    return lax.broadcasted_iota(_i32, (1, 128), 1)


def _get(vec, j):
    return jnp.sum(jnp.where(_col128() == j, vec, jnp.zeros_like(vec)))


def _put(acc, j, val):
    v = jnp.broadcast_to(jnp.asarray(val, acc.dtype), acc.shape)
    return jnp.where(_col128() == j, v, acc)


def _tc_me0_body(part_ref, ppart_ref, ranks_ref, etgt_ref, escal_ref):
    h3 = part_ref[...]
    h = jnp.sum(h3, axis=0)
    lane_cum = _cumsum_lanes(h)
    rowtot = lane_cum[:, 127:128]
    rowoff = _cumsum_rows(rowtot) - rowtot
    cum = lane_cum + rowoff
    binidx = (lax.broadcasted_iota(_i32, (32, 128), 0) * 128
              + lax.broadcasted_iota(_i32, (32, 128), 1))
    ranks = ranks_ref[...]
    etgt = jnp.zeros((1, 128), _i32)
    for j in range(18):
        r = _get(ranks, j)
        B = jnp.sum((cum <= r).astype(_i32))
        at = binidx == B
        hB = jnp.sum(jnp.where(at, h, 0))
        cB = jnp.sum(jnp.where(at, cum, 0))
        etgt = _put(etgt, j, B)
        etgt = _put(etgt, 32 + j, r - (cB - hB))
    etgt_ref[...] = etgt
    S = jnp.sum(ppart_ref[...])
    fb = S <= np.float32(1e-8)
    escal = jnp.zeros((1, 128), _f32)
    escal = _put(escal, 0, jnp.where(fb, np.float32(0.0),
                                     np.float32(1.0) / S))
    escal = _put(escal, 1, jnp.where(fb, np.float32(1.0) / np.float32(N),
                                     np.float32(0.0)))
    escal = _put(escal, 2, jnp.where(fb, np.float32(0.0),
                                     np.float32(1.0)))
    escal = _put(escal, 3, jnp.where(fb,
                                     jnp.log(np.float32(1.0) / np.float32(N)),
                                     -jnp.log(S)))
    escal_ref[...] = escal


def _tc_me0(part, ppart):
    return pl.pallas_call(
        _tc_me0_body,
        out_shape=[jax.ShapeDtypeStruct((1, 128), _i32),
                   jax.ShapeDtypeStruct((1, 128), _f32)],
    )(part, ppart, _consts()["ranks_op"])


def _seg_cum_256(h):
    lane_cum = _cumsum_lanes(h)
    rowtot = lane_cum[:, 127:128]
    prev = pltpu.roll(rowtot, 1, axis=0)
    row = lax.broadcasted_iota(_i32, lane_cum.shape, 0)
    odd = (row % 2) == 1
    return lane_cum + jnp.where(odd, prev, jnp.zeros_like(prev))


def _owner18(prev, j):
    pj = _get(prev, j)
    owner = jnp.asarray(j, _i32)
    for t2 in range(18):
        owner = jnp.where(_get(prev, t2) == pj, jnp.asarray(t2, _i32), owner)
    return owner


def _tc_me12_body(part_ref, prev_ref, etgt_ref, *, shift_out):
    h3 = part_ref[...]
    h = jnp.sum(h3, axis=0)
    cum = _seg_cum_256(h)
    row = lax.broadcasted_iota(_i32, (36, 128), 0)
    sub = ((row % 2) * 128 + lax.broadcasted_iota(_i32, (36, 128), 1))
    tgt = row // 2
    prev = prev_ref[...]
    etgt = jnp.zeros((1, 128), _i32)
    for j in range(18):
        r = _get(prev, 32 + j)
        pref = _get(prev, j)
        mine = tgt == _owner18(prev, j)
        B = jnp.sum((mine & (cum <= r)).astype(_i32))
        at = mine & (sub == B)
        hB = jnp.sum(jnp.where(at, h, 0))
        cB = jnp.sum(jnp.where(at, cum, 0))
        etgt = _put(etgt, j, (pref << shift_out) | B)
        etgt = _put(etgt, 32 + j, r - (cB - hB))
    etgt_ref[...] = etgt


def _tc_me12(part, prev, shift_out):
    return pl.pallas_call(
        functools.partial(_tc_me12_body, shift_out=shift_out),
        out_shape=jax.ShapeDtypeStruct((1, 128), _i32),
    )(part, prev)


def _tc_me3_body(part_ref, prev_ref, wts_ref, qb_ref):
    h3 = part_ref[...]
    h = jnp.sum(h3, axis=0)
    col = lax.broadcasted_iota(_i32, (3, 128), 1)
    cum = h
    for k in (1, 2, 4, 8):
        sh = pltpu.roll(cum, k, axis=1)
        cum = cum + jnp.where((col % 16) >= k, sh, jnp.zeros_like(sh))
    row = lax.broadcasted_iota(_i32, (3, 128), 0)
    flat = row * 128 + col
    tgt = flat // 16
    prev = prev_ref[...]
    wts = wts_ref[...]
    vals = []
    for j in range(18):
        r = _get(prev, 32 + j)
        pref = _get(prev, j)
        mine = tgt == _owner18(prev, j)
        B0 = jnp.sum((mine & (cum <= r)).astype(_i32))
        key = ((pref.astype(_u32) << np.uint32(4)) | B0.astype(_u32))
        bits = jnp.where((key & np.uint32(0x80000000)) != 0,
                         key ^ np.uint32(0x80000000), ~key)
        vals.append(lax.bitcast_convert_type(bits, _f32))
    qb = jnp.zeros((1, 128), _f32)
    for k in range(9):
        qb = _put(qb, k, vals[2 * k] * _get(wts, k)
                  + vals[2 * k + 1] * _get(wts, 32 + k))
    qb_ref[...] = qb


def _tc_me3(part, prev):
    return pl.pallas_call(
        _tc_me3_body,
        out_shape=jax.ShapeDtypeStruct((1, 128), _f32),
    )(part, prev, _consts()["wts_op"])


def _tc_skey_body(e_ref, s_ref, g_ref, qb_ref, escal_ref, skey_ref, strat_ref):
    e = e_ref[...]
    s = s_ref[...]
    escal = escal_ref[...]
    qb = qb_ref[...]
    A = _get(escal, 2)
    C = _get(escal, 3)
    beta = np.float32(1.0) + np.float32(0.025) * s.astype(_f32)
    be = beta * jnp.clip(e, -10.0, 10.0)
    score = jnp.maximum(A * be + C, _LOG1EM8) + g_ref[...]
    skey_ref[...] = _mono_u32(score)
    strat = jnp.zeros_like(s)
    for k in range(9):
        strat = strat + (_get(qb, k) <= e).astype(_i32)
    strat_ref[...] = jnp.where(_valid2d(), strat, NUM_STRATA)


def _tc_skey(e2, s2, qb, escal):
    return pl.pallas_call(
        _tc_skey_body,
        out_shape=[jax.ShapeDtypeStruct((ROWS, 128), _u32),
                   jax.ShapeDtypeStruct((ROWS, 128), _i32)],
    )(e2, s2, _consts()["gumbel2d"], qb, escal)


def _tc_ms_body(part_ref, prev_ref, bscal_ref, stgt_ref, *, level):
    h3 = part_ref[...]
    h = jnp.sum(h3, axis=0)
    cum = _seg_cum_256(h)
    row = lax.broadcasted_iota(_i32, (20, 128), 0)
    sub = ((row % 2) * 128 + lax.broadcasted_iota(_i32, (20, 128), 1))
    strat = row // 2
    prev = prev_ref[...]
    bscal = bscal_ref[...]
    stgt = jnp.zeros((1, 128), _i32)
    for i in range(NUM_STRATA):
        mine = strat == i
        tot = jnp.sum(jnp.where(mine, h, 0))
        if level == 0:
            q1 = _get(bscal, 0)
            q2 = _get(bscal, 1)
            b = jnp.minimum(jnp.maximum(1, (tot * q1) // q2), tot)
            need = b
            pref = jnp.asarray(0, _i32)
        else:
            need = _get(prev, 16 + i)
            pref = _get(prev, i)
            b = _get(prev, 32 + i)
        suf = tot - cum + h
        Bv = jnp.sum((mine & (suf >= need)).astype(_i32)) - 1
        at = mine & (sub == Bv)
        hB = jnp.sum(jnp.where(at, h, 0))
        sB = jnp.sum(jnp.where(at, suf, 0))
        dead = b <= 0
        stgt = _put(stgt, i, jnp.where(dead, _SENT << (8 * level),
                                       (pref << 8) | jnp.maximum(Bv, 0)))
        stgt = _put(stgt, 16 + i, jnp.where(dead, 0, need - (sB - hB)))
        stgt = _put(stgt, 32 + i, b)
    stgt = _put(stgt, NUM_STRATA, _SENT << (8 * level))
    stgt_ref[...] = stgt


def _tc_ms(part, prev, bscal, level):
    return pl.pallas_call(
        functools.partial(_tc_ms_body, level=level),
        out_shape=jax.ShapeDtypeStruct((1, 128), _i32),
    )(part, prev, bscal)


def _tc_ms3_body(part_ref, prev_ref, tsel_ref):
    h3 = part_ref[...]
    h = jnp.sum(h3, axis=0)
    cum = _seg_cum_256(h)
    row = lax.broadcasted_iota(_i32, (20, 128), 0)
    strat = row // 2
    prev = prev_ref[...]
    tsel = jnp.full((1, 128), np.uint32(0xFFFFFFFF), _u32)
    for i in range(NUM_STRATA):
        mine = strat == i
        need = _get(prev, 16 + i)
        pref = _get(prev, i)
        b = _get(prev, 32 + i)
        tot = jnp.sum(jnp.where(mine, h, 0))
        suf = tot - cum + h
        Bv = jnp.sum((mine & (suf >= need)).astype(_i32)) - 1
        T = ((pref.astype(_u32) << np.uint32(8))
             | jnp.maximum(Bv, 0).astype(_u32))
        tsel = _put(tsel, i, jnp.where(b <= 0, np.uint32(0xFFFFFFFF), T))
    tsel_ref[...] = tsel


def _tc_ms3(part, prev):
    return pl.pallas_call(
        _tc_ms3_body,
        out_shape=jax.ShapeDtypeStruct((1, 128), _u32),
    )(part, prev)


def _tc_final_body(e_ref, s_ref, skey_ref, strat_ref, tsel_ref, escal_ref,
                   probs_ref, sel_ref):
    e = e_ref[...]
    s = s_ref[...]
    escal = escal_ref[...]
    tsel = tsel_ref[...]
    beta = np.float32(1.0) + np.float32(0.025) * s.astype(_f32)
    p = jnp.exp(beta * jnp.clip(e, -10.0, 10.0))
    probs_ref[...] = p * _get(escal, 0) + _get(escal, 1)
    strat = strat_ref[...]
    tsel_i = lax.bitcast_convert_type(tsel, _i32)
    T = jnp.full(strat.shape, np.uint32(0xFFFFFFFF), _u32)
    for i in range(NUM_STRATA):
        Ti = lax.bitcast_convert_type(_get(tsel_i, i), _u32)
        T = jnp.where(strat == i, Ti, T)
    sel_ref[...] = (skey_ref[...] >= T).astype(_i32)


def _tc_final(e2, s2, skey2, strat2, tsel, escal):
    return pl.pallas_call(
        _tc_final_body,
        out_shape=[jax.ShapeDtypeStruct((ROWS, 128), _f32),
                   jax.ShapeDtypeStruct((ROWS, 128), _i32)],
    )(e2, s2, skey2, strat2, tsel, escal)


_MESH = plsc.VectorSubcoreMesh(core_axis_name="c", subcore_axis_name="s",
                               num_cores=2, num_subcores=16)


def _zero_ref(ref, nwords):
    z = jnp.zeros((16,), _i32)

    def body(j, carry):
        ref[pl.ds(j * 16, 16)] = z
        return carry

    lax.fori_loop(0, nwords // 16, body, 0)


def _fold_lanes(hist, fold, nbins):
    iota = lax.broadcasted_iota(_i32, (16,), 0)

    def body(j, carry):
        bins = iota + j * 16
        acc = jnp.zeros((16,), _i32)
        for L in range(16):
            acc = acc + plsc.load_gather(hist, [bins * 16 + L])
        fold[pl.ds(j * 16, 16)] = acc
        return carry

    lax.fori_loop(0, nbins // 16, body, 0)


def _sc_wid():
    return lax.axis_index("s") * 2 + lax.axis_index("c")


def _sc_sweep(kbuf_list, nv, per_vec):
    def body(i, carry):
        vecs = [b[pl.ds(i * 16, 16)] for b in kbuf_list]
        per_vec(i, *vecs)
        return carry

    lax.fori_loop(0, nv, body, 0)


def _sc_ehist_body(ekey_ref, part_ref, kbuf, hist, fold, *, level, nbins,
                   outb, histwords, tgt_ref=None, tbuf=None):
    wid = _sc_wid()
    base = wid * (NVT * 16)
    _zero_ref(hist, histwords)
    iota = lax.broadcasted_iota(_i32, (16,), 0)
    ones = jnp.full((16,), 1, _i32)

    if level > 0:
        pltpu.sync_copy(tgt_ref, tbuf)
        t0 = tbuf[pl.ds(0, 16)]
        t1 = tbuf[pl.ds(16, 16)]
        prefs = [t0[t] for t in range(16)] + [t1[0], t1[1]]
        shift = {1: 20, 2: 12, 3: 4}[level]
        submask = np.uint32(0xF if level == 3 else 0xFF)
        subshift = {1: 12, 2: 4, 3: 0}[level]
        nsub = 16 if level == 3 else 256

    off = 0
    for nv in BLOCKS:
        pltpu.sync_copy(ekey_ref.at[pl.ds(base + off * 16, nv * 16)],
                        kbuf.at[pl.ds(0, nv * 16)])

        if level == 0:
            def per_vec(i, kv):
                bin12 = (kv >> np.uint32(20)).astype(_i32)
                plsc.addupdate_scatter(hist, [bin12 * 16 + iota], ones)
        else:
            def per_vec(i, kv):
                v = (kv >> np.uint32(shift)).astype(_i32)
                pos = jnp.zeros((16,), _i32)
                for t in range(18):
                    pos = pos + (prefs[t] <= v).astype(_i32)
                slot = jnp.maximum(pos - 1, 0)
                pref = plsc.load_gather(tbuf, [slot])
                match = (pos > 0) & (pref == v)
                sub = ((kv >> np.uint32(subshift)) & submask).astype(_i32)
                idx = (slot * nsub + sub) * 16 + iota
                plsc.addupdate_scatter(hist, [idx], ones, mask=match)

        _sc_sweep([kbuf], nv, per_vec)
        off += nv

    _zero_ref(fold, outb)
    _fold_lanes(hist, fold, nbins)
    pltpu.sync_copy(fold, part_ref.at[wid])


def _make_sc_ehist(level):
    nbins = {0: 4096, 1: 4608, 2: 4608, 3: 288}[level]
    outb = {0: 4096, 1: 4608, 2: 4608, 3: 384}[level]
    histwords = {0: 65536, 1: 73728, 2: 73728, 3: 6144}[level]
    scratch = [pltpu.VMEM((8192,), _u32),
               pltpu.VMEM((histwords,), _i32),
               pltpu.VMEM((outb,), _i32)]
    if level > 0:
        scratch += [pltpu.VMEM((128,), _i32)]

    def body(*args):
        if level == 0:
            ekey_ref, part_ref, kbuf, hist, fold = args
            _sc_ehist_body(ekey_ref, part_ref, kbuf, hist, fold, level=0,
                           nbins=nbins, outb=outb, histwords=histwords)
        else:
            ekey_ref, tgt_ref, part_ref, kbuf, hist, fold, tbuf = args
            _sc_ehist_body(ekey_ref, part_ref, kbuf, hist, fold, level=level,
                           nbins=nbins, outb=outb, histwords=histwords,
                           tgt_ref=tgt_ref, tbuf=tbuf)

    return pl.kernel(
        body,
        out_type=jax.ShapeDtypeStruct((NT, outb), _i32),
        mesh=_MESH,
        compiler_params=pltpu.CompilerParams(needs_layout_passes=False),
        scratch_types=scratch,
    )


def _sc_shist_body(skey_ref, strat_ref, part_ref, kbuf, sbuf, hist, fold,
                   *, level, tgt_ref=None, tbuf=None):
    wid = _sc_wid()
    base = wid * (NVT * 16)
    _zero_ref(hist, 45056)
    iota = lax.broadcasted_iota(_i32, (16,), 0)
    ones = jnp.full((16,), 1, _i32)

    if level > 0:
        pltpu.sync_copy(tgt_ref, tbuf)
        shift = {1: 24, 2: 16, 3: 8}[level]
        subshift = {1: 16, 2: 8, 3: 0}[level]

    off = 0
    for nv in BLOCKS:
        pltpu.sync_copy(skey_ref.at[pl.ds(base + off * 16, nv * 16)],
                        kbuf.at[pl.ds(0, nv * 16)])
        pltpu.sync_copy(strat_ref.at[pl.ds(base + off * 16, nv * 16)],
                        sbuf.at[pl.ds(0, nv * 16)])

        if level == 0:
            def per_vec(i, kv, st):
                sub = (kv >> np.uint32(24)).astype(_i32)
                plsc.addupdate_scatter(hist, [(st * 256 + sub) * 16 + iota],
                                       ones)
        else:
            def per_vec(i, kv, st):
                pref = plsc.load_gather(tbuf, [st])
                match = (kv >> np.uint32(shift)).astype(_i32) == pref
                sub = ((kv >> np.uint32(subshift)) & np.uint32(0xFF)) \
                    .astype(_i32)
                idx = (st * 256 + sub) * 16 + iota
                plsc.addupdate_scatter(hist, [idx], ones, mask=match)

        _sc_sweep([kbuf, sbuf], nv, per_vec)
        off += nv

    _fold_lanes(hist, fold, 2560)
    pltpu.sync_copy(fold, part_ref.at[wid])


def _make_sc_shist(level):
    scratch = [pltpu.VMEM((8192,), _u32),
               pltpu.VMEM((8192,), _i32),
               pltpu.VMEM((45056,), _i32),
               pltpu.VMEM((2560,), _i32)]
    if level > 0:
        scratch += [pltpu.VMEM((128,), _i32)]

    def body(*args):
        if level == 0:
            skey_ref, strat_ref, part_ref, kbuf, sbuf, hist, fold = args
            _sc_shist_body(skey_ref, strat_ref, part_ref, kbuf, sbuf, hist,
                           fold, level=0)
        else:
            (skey_ref, strat_ref, tgt_ref, part_ref, kbuf, sbuf, hist, fold,
             tbuf) = args
            _sc_shist_body(skey_ref, strat_ref, part_ref, kbuf, sbuf, hist,
                           fold, level=level, tgt_ref=tgt_ref, tbuf=tbuf)

    return pl.kernel(
        body,
        out_type=jax.ShapeDtypeStruct((NT, 2560), _i32),
        mesh=_MESH,
        compiler_params=pltpu.CompilerParams(needs_layout_passes=False),
        scratch_types=scratch,
    )


_SC_EHIST = {lvl: _make_sc_ehist(lvl) for lvl in range(4)}
_SC_SHIST = {lvl: _make_sc_shist(lvl) for lvl in range(4)}


def kernel(entropy_map, node_scales, budget):
    e2 = jnp.pad(entropy_map, (0, PAD - N),
                 constant_values=np.float32(np.inf)).reshape(ROWS, 128)
    s2 = jnp.pad(node_scales.astype(_i32), (0, PAD - N)).reshape(ROWS, 128)

    ekey2, ppart = _tc_ekey(e2, s2)
    ekey1 = ekey2.reshape(PAD)

    eh0 = _SC_EHIST[0](ekey1).reshape(NT, 32, 128)
    etgt, escal = _tc_me0(eh0, ppart)
    eh1 = _SC_EHIST[1](ekey1, etgt.reshape(128)).reshape(NT, 36, 128)
    etgt = _tc_me12(eh1, etgt, 8)
    eh2 = _SC_EHIST[2](ekey1, etgt.reshape(128)).reshape(NT, 36, 128)
    etgt = _tc_me12(eh2, etgt, 8)
    eh3 = _SC_EHIST[3](ekey1, etgt.reshape(128)).reshape(NT, 3, 128)
    qb = _tc_me3(eh3, etgt)

    skey2, strat2 = _tc_skey(e2, s2, qb, escal)
    skey1 = skey2.reshape(PAD)
    strat1 = strat2.reshape(PAD)

    budget = jnp.asarray(budget, _i32)
    g = jnp.gcd(budget, N)
    bscal = jnp.zeros((1, 128), _i32)
    bscal = bscal.at[0, 0].set(budget // g).at[0, 1].set(N // g)

    sh0 = _SC_SHIST[0](skey1, strat1).reshape(NT, 20, 128)
    stgt = _tc_ms(sh0, bscal, bscal, level=0)
    sh1 = _SC_SHIST[1](skey1, strat1, stgt.reshape(128)).reshape(NT, 20, 128)
    stgt = _tc_ms(sh1, stgt, bscal, level=1)
    sh2 = _SC_SHIST[2](skey1, strat1, stgt.reshape(128)).reshape(NT, 20, 128)
    stgt = _tc_ms(sh2, stgt, bscal, level=2)
    sh3 = _SC_SHIST[3](skey1, strat1, stgt.reshape(128)).reshape(NT, 20, 128)
    tsel = _tc_ms3(sh3, stgt)

    probs2, sel2 = _tc_final(e2, s2, skey2, strat2, tsel, escal)
    probs = probs2.reshape(PAD)[:N]
    mask = sel2.reshape(PAD)[:N].astype(jnp.bool_)
    return probs, mask


# speedup vs baseline: 17.5135x; 1.0213x over previous
import functools

import numpy as np
import jax
import jax.numpy as jnp
from jax import lax
from jax.experimental import pallas as pl
from jax.experimental.pallas import tpu as pltpu
from jax.experimental.pallas import tpu_sc as plsc

N = 1_000_000
PAD = 1_000_448
ROWS = PAD // 128
NT = 32
NVT = 1954
BLOCKS = (512, 512, 512, 418)
NUM_STRATA = 10

_i32 = jnp.int32
_u32 = jnp.uint32
_f32 = jnp.float32


_QQ = np.array([0, 1203982323, 1212370931, 1217559543, 1220759539,
                1223959536, 1225948151, 1227548149, 1229148147, 1230748146,
                1232348144], dtype=np.uint32).view(np.float32)
_low9 = np.floor(_QQ).astype(np.int32)[1:10]
_high9 = np.ceil(_QQ).astype(np.int32)[1:10]
_hw9 = (_QQ - np.floor(_QQ)).astype(np.float32)[1:10]
_lw9 = (np.float32(1.0) - _hw9).astype(np.float32)
_RANKS18 = np.empty(18, np.int32)
_RANKS18[0::2] = _low9
_RANKS18[1::2] = _high9
_ranks_np = np.zeros((1, 128), np.int32)
_ranks_np[0, :18] = _RANKS18
_wts_np = np.zeros((1, 128), np.float32)
_wts_np[0, :9] = _lw9
_wts_np[0, 32:41] = _hw9

_CONSTS = {}


def _consts():
    if _CONSTS:
        return _CONSTS
    u = jax.random.uniform(jax.random.key(42), (N,), minval=1e-12, maxval=1.0)
    g = jnp.pad(-jnp.log(-jnp.log(u)), (0, PAD - N)).reshape(ROWS, 128)
    out = {"gumbel2d": g,
           "ranks_op": jnp.asarray(_ranks_np),
           "wts_op": jnp.asarray(_wts_np)}
    if not isinstance(g, jax.core.Tracer):
        _CONSTS.update(out)
    return out

_LOG1EM8 = np.float32(np.log(np.float32(1e-8)))
_SENT = 0x1FF


def _mono_u32(x):
    b = lax.bitcast_convert_type(x, _u32)
    neg = (b & np.uint32(0x80000000)) != 0
    return jnp.where(neg, ~b, b | np.uint32(0x80000000))


def _valid2d():
    r = lax.broadcasted_iota(_i32, (ROWS, 128), 0)
    c = lax.broadcasted_iota(_i32, (ROWS, 128), 1)
    return (r * 128 + c) < N


def _tc_ekey_body(e_ref, s_ref, ekey_ref, ppart_ref):
    e = e_ref[...]
    s = s_ref[...]
    ekey_ref[...] = _mono_u32(e)
    beta = np.float32(1.0) + np.float32(0.025) * s.astype(_f32)
    p = jnp.exp(beta * jnp.clip(e, -10.0, 10.0))
    p = jnp.where(_valid2d(), p, np.float32(0.0))
    ppart_ref[...] = jnp.sum(p, axis=0, keepdims=True)


def _tc_ekey(e2, s2):
    return pl.pallas_call(
        _tc_ekey_body,
        out_shape=[jax.ShapeDtypeStruct((ROWS, 128), _u32),
                   jax.ShapeDtypeStruct((1, 128), _f32)],
    )(e2, s2)


def _cumsum_lanes(x):
    col = lax.broadcasted_iota(_i32, x.shape, 1)
    for k in (1, 2, 4, 8, 16, 32, 64):
        sh = pltpu.roll(x, k, axis=1)
        x = x + jnp.where(col >= k, sh, jnp.zeros_like(x))
    return x


def _cumsum_rows(x):
    row = lax.broadcasted_iota(_i32, x.shape, 0)
    k = 1
    while k < x.shape[0]:
        sh = pltpu.roll(x, k, axis=0)
        x = x + jnp.where(row >= k, sh, jnp.zeros_like(x))
        k *= 2
    return x


_COL128 = None


def _col128():
    return lax.broadcasted_iota(_i32, (1, 128), 1)


def _get(vec, j):
    return jnp.sum(jnp.where(_col128() == j, vec, jnp.zeros_like(vec)))


def _put(acc, j, val):
    v = jnp.broadcast_to(jnp.asarray(val, acc.dtype), acc.shape)
    return jnp.where(_col128() == j, v, acc)


def _tc_me0_body(part_ref, ppart_ref, ranks_ref, etgt_ref, escal_ref):
    h3 = part_ref[...]
    h = jnp.sum(h3, axis=0)
    lane_cum = _cumsum_lanes(h)
    rowtot = lane_cum[:, 127:128]
    rowoff = _cumsum_rows(rowtot) - rowtot
    cum = lane_cum + rowoff
    binidx = (lax.broadcasted_iota(_i32, (32, 128), 0) * 128
              + lax.broadcasted_iota(_i32, (32, 128), 1))
    ranks = ranks_ref[...]
    etgt = jnp.zeros((1, 128), _i32)
    for j in range(18):
        r = _get(ranks, j)
        B = jnp.sum((cum <= r).astype(_i32))
        at = binidx == B
        hB = jnp.sum(jnp.where(at, h, 0))
        cB = jnp.sum(jnp.where(at, cum, 0))
        etgt = _put(etgt, j, B)
        etgt = _put(etgt, 32 + j, r - (cB - hB))
    col = _col128()
    etgt = jnp.where((col >= 18) & (col < 32),
                     jnp.full_like(etgt, np.int32(0x7FFFFFFF)), etgt)
    etgt_ref[...] = etgt
    S = jnp.sum(ppart_ref[...])
    fb = S <= np.float32(1e-8)
    escal = jnp.zeros((1, 128), _f32)
    escal = _put(escal, 0, jnp.where(fb, np.float32(0.0),
                                     np.float32(1.0) / S))
    escal = _put(escal, 1, jnp.where(fb, np.float32(1.0) / np.float32(N),
                                     np.float32(0.0)))
    escal = _put(escal, 2, jnp.where(fb, np.float32(0.0),
                                     np.float32(1.0)))
    escal = _put(escal, 3, jnp.where(fb,
                                     jnp.log(np.float32(1.0) / np.float32(N)),
                                     -jnp.log(S)))
    escal_ref[...] = escal


def _tc_me0(part, ppart):
    return pl.pallas_call(
        _tc_me0_body,
        out_shape=[jax.ShapeDtypeStruct((1, 128), _i32),
                   jax.ShapeDtypeStruct((1, 128), _f32)],
    )(part, ppart, _consts()["ranks_op"])


def _seg_cum_256(h):
    lane_cum = _cumsum_lanes(h)
    rowtot = lane_cum[:, 127:128]
    prev = pltpu.roll(rowtot, 1, axis=0)
    row = lax.broadcasted_iota(_i32, lane_cum.shape, 0)
    odd = (row % 2) == 1
    return lane_cum + jnp.where(odd, prev, jnp.zeros_like(prev))


def _owner18(prev, j):
    pj = _get(prev, j)
    owner = jnp.asarray(j, _i32)
    for t2 in range(18):
        owner = jnp.where(_get(prev, t2) == pj, jnp.asarray(t2, _i32), owner)
    return owner


def _tc_me12_body(part_ref, prev_ref, etgt_ref, *, shift_out):
    h3 = part_ref[...]
    h = jnp.sum(h3, axis=0)
    cum = _seg_cum_256(h)
    row = lax.broadcasted_iota(_i32, (36, 128), 0)
    sub = ((row % 2) * 128 + lax.broadcasted_iota(_i32, (36, 128), 1))
    tgt = row // 2
    prev = prev_ref[...]
    etgt = jnp.zeros((1, 128), _i32)
    for j in range(18):
        r = _get(prev, 32 + j)
        pref = _get(prev, j)
        mine = tgt == _owner18(prev, j)
        B = jnp.sum((mine & (cum <= r)).astype(_i32))
        at = mine & (sub == B)
        hB = jnp.sum(jnp.where(at, h, 0))
        cB = jnp.sum(jnp.where(at, cum, 0))
        etgt = _put(etgt, j, (pref << shift_out) | B)
        etgt = _put(etgt, 32 + j, r - (cB - hB))
    col = _col128()
    etgt = jnp.where((col >= 18) & (col < 32),
                     jnp.full_like(etgt, np.int32(0x7FFFFFFF)), etgt)
    etgt_ref[...] = etgt


def _tc_me12(part, prev, shift_out):
    return pl.pallas_call(
        functools.partial(_tc_me12_body, shift_out=shift_out),
        out_shape=jax.ShapeDtypeStruct((1, 128), _i32),
    )(part, prev)


def _tc_me3_body(part_ref, prev_ref, wts_ref, qb_ref):
    h3 = part_ref[...]
    h = jnp.sum(h3, axis=0)
    col = lax.broadcasted_iota(_i32, (3, 128), 1)
    cum = h
    for k in (1, 2, 4, 8):
        sh = pltpu.roll(cum, k, axis=1)
        cum = cum + jnp.where((col % 16) >= k, sh, jnp.zeros_like(sh))
    row = lax.broadcasted_iota(_i32, (3, 128), 0)
    flat = row * 128 + col
    tgt = flat // 16
    prev = prev_ref[...]
    wts = wts_ref[...]
    vals = []
    for j in range(18):
        r = _get(prev, 32 + j)
        pref = _get(prev, j)
        mine = tgt == _owner18(prev, j)
        B0 = jnp.sum((mine & (cum <= r)).astype(_i32))
        key = ((pref.astype(_u32) << np.uint32(4)) | B0.astype(_u32))
        bits = jnp.where((key & np.uint32(0x80000000)) != 0,
                         key ^ np.uint32(0x80000000), ~key)
        vals.append(lax.bitcast_convert_type(bits, _f32))
    qb = jnp.zeros((1, 128), _f32)
    for k in range(9):
        qb = _put(qb, k, vals[2 * k] * _get(wts, k)
                  + vals[2 * k + 1] * _get(wts, 32 + k))
    qb_ref[...] = qb


def _tc_me3(part, prev):
    return pl.pallas_call(
        _tc_me3_body,
        out_shape=jax.ShapeDtypeStruct((1, 128), _f32),
    )(part, prev, _consts()["wts_op"])


def _tc_skey_body(e_ref, s_ref, g_ref, qb_ref, escal_ref, skey_ref, strat_ref):
    e = e_ref[...]
    s = s_ref[...]
    escal = escal_ref[...]
    qb = qb_ref[...]
    A = _get(escal, 2)
    C = _get(escal, 3)
    beta = np.float32(1.0) + np.float32(0.025) * s.astype(_f32)
    be = beta * jnp.clip(e, -10.0, 10.0)
    score = jnp.maximum(A * be + C, _LOG1EM8) + g_ref[...]
    skey_ref[...] = _mono_u32(score)
    strat = jnp.zeros_like(s)
    for k in range(9):
        strat = strat + (_get(qb, k) <= e).astype(_i32)
    strat_ref[...] = jnp.where(_valid2d(), strat, NUM_STRATA)


def _tc_skey(e2, s2, qb, escal):
    return pl.pallas_call(
        _tc_skey_body,
        out_shape=[jax.ShapeDtypeStruct((ROWS, 128), _u32),
                   jax.ShapeDtypeStruct((ROWS, 128), _i32)],
    )(e2, s2, _consts()["gumbel2d"], qb, escal)


def _tc_ms_body(part_ref, prev_ref, bscal_ref, stgt_ref, *, level):
    h3 = part_ref[...]
    h = jnp.sum(h3, axis=0)
    cum = _seg_cum_256(h)
    row = lax.broadcasted_iota(_i32, (20, 128), 0)
    sub = ((row % 2) * 128 + lax.broadcasted_iota(_i32, (20, 128), 1))
    strat = row // 2
    prev = prev_ref[...]
    bscal = bscal_ref[...]
    stgt = jnp.zeros((1, 128), _i32)
    for i in range(NUM_STRATA):
        mine = strat == i
        tot = jnp.sum(jnp.where(mine, h, 0))
        if level == 0:
            q1 = _get(bscal, 0)
            q2 = _get(bscal, 1)
            b = jnp.minimum(jnp.maximum(1, (tot * q1) // q2), tot)
            need = b
            pref = jnp.asarray(0, _i32)
        else:
            need = _get(prev, 16 + i)
            pref = _get(prev, i)
            b = _get(prev, 32 + i)
        suf = tot - cum + h
        Bv = jnp.sum((mine & (suf >= need)).astype(_i32)) - 1
        at = mine & (sub == Bv)
        hB = jnp.sum(jnp.where(at, h, 0))
        sB = jnp.sum(jnp.where(at, suf, 0))
        dead = b <= 0
        stgt = _put(stgt, i, jnp.where(dead, _SENT << (8 * level),
                                       (pref << 8) | jnp.maximum(Bv, 0)))
        stgt = _put(stgt, 16 + i, jnp.where(dead, 0, need - (sB - hB)))
        stgt = _put(stgt, 32 + i, b)
    stgt = _put(stgt, NUM_STRATA, _SENT << (8 * level))
    stgt_ref[...] = stgt


def _tc_ms(part, prev, bscal, level):
    return pl.pallas_call(
        functools.partial(_tc_ms_body, level=level),
        out_shape=jax.ShapeDtypeStruct((1, 128), _i32),
    )(part, prev, bscal)


def _tc_ms3_body(part_ref, prev_ref, tsel_ref):
    h3 = part_ref[...]
    h = jnp.sum(h3, axis=0)
    cum = _seg_cum_256(h)
    row = lax.broadcasted_iota(_i32, (20, 128), 0)
    strat = row // 2
    prev = prev_ref[...]
    tsel = jnp.full((1, 128), np.uint32(0xFFFFFFFF), _u32)
    for i in range(NUM_STRATA):
        mine = strat == i
        need = _get(prev, 16 + i)
        pref = _get(prev, i)
        b = _get(prev, 32 + i)
        tot = jnp.sum(jnp.where(mine, h, 0))
        suf = tot - cum + h
        Bv = jnp.sum((mine & (suf >= need)).astype(_i32)) - 1
        T = ((pref.astype(_u32) << np.uint32(8))
             | jnp.maximum(Bv, 0).astype(_u32))
        tsel = _put(tsel, i, jnp.where(b <= 0, np.uint32(0xFFFFFFFF), T))
    tsel_ref[...] = tsel


def _tc_ms3(part, prev):
    return pl.pallas_call(
        _tc_ms3_body,
        out_shape=jax.ShapeDtypeStruct((1, 128), _u32),
    )(part, prev)


def _tc_final_body(e_ref, s_ref, skey_ref, strat_ref, tsel_ref, escal_ref,
                   probs_ref, sel_ref):
    e = e_ref[...]
    s = s_ref[...]
    escal = escal_ref[...]
    tsel = tsel_ref[...]
    beta = np.float32(1.0) + np.float32(0.025) * s.astype(_f32)
    p = jnp.exp(beta * jnp.clip(e, -10.0, 10.0))
    probs_ref[...] = p * _get(escal, 0) + _get(escal, 1)
    strat = strat_ref[...]
    tsel_i = lax.bitcast_convert_type(tsel, _i32)
    T = jnp.full(strat.shape, np.uint32(0xFFFFFFFF), _u32)
    for i in range(NUM_STRATA):
        Ti = lax.bitcast_convert_type(_get(tsel_i, i), _u32)
        T = jnp.where(strat == i, Ti, T)
    sel_ref[...] = (skey_ref[...] >= T).astype(_i32)


def _tc_final(e2, s2, skey2, strat2, tsel, escal):
    return pl.pallas_call(
        _tc_final_body,
        out_shape=[jax.ShapeDtypeStruct((ROWS, 128), _f32),
                   jax.ShapeDtypeStruct((ROWS, 128), _i32)],
    )(e2, s2, skey2, strat2, tsel, escal)


_MESH = plsc.VectorSubcoreMesh(core_axis_name="c", subcore_axis_name="s",
                               num_cores=2, num_subcores=16)


def _zero_ref(ref, nwords):
    z = jnp.zeros((16,), _i32)
    assert nwords % 128 == 0

    def body(j, carry):
        for u in range(8):
            ref[pl.ds(j * 128 + u * 16, 16)] = z
        return carry

    lax.fori_loop(0, nwords // 128, body, 0)


def _fold_lanes(hist, fold, nbins):
    iota = lax.broadcasted_iota(_i32, (16,), 0)
    assert nbins % 32 == 0

    def body(j, carry):
        for u in range(2):
            bins = iota + (j * 32 + u * 16)
            acc = jnp.zeros((16,), _i32)
            for L in range(16):
                acc = acc + plsc.load_gather(hist, [bins * 16 + L])
            fold[pl.ds(j * 32 + u * 16, 16)] = acc
        return carry

    lax.fori_loop(0, nbins // 32, body, 0)


def _sc_wid():
    return lax.axis_index("s") * 2 + lax.axis_index("c")


def _sc_sweep(kbuf_list, nv, per_vec):
    n4 = nv // 4

    def body(i, carry):
        for u in range(4):
            j = i * 4 + u
            vecs = [b[pl.ds(j * 16, 16)] for b in kbuf_list]
            per_vec(j, *vecs)
        return carry

    lax.fori_loop(0, n4, body, 0)
    for j in range(n4 * 4, nv):
        vecs = [b[pl.ds(j * 16, 16)] for b in kbuf_list]
        per_vec(j, *vecs)


def _sc_ehist_body(ekey_ref, part_ref, kbuf, hist, fold, *, level, nbins,
                   outb, histwords, tgt_ref=None, tbuf=None):
    wid = _sc_wid()
    base = wid * (NVT * 16)
    _zero_ref(hist, histwords)
    iota = lax.broadcasted_iota(_i32, (16,), 0)
    ones = jnp.full((16,), 1, _i32)

    if level > 0:
        pltpu.sync_copy(tgt_ref, tbuf)
        shift = {1: 20, 2: 12, 3: 4}[level]
        submask = np.uint32(0xF if level == 3 else 0xFF)
        subshift = {1: 12, 2: 4, 3: 0}[level]
        nsub = 16 if level == 3 else 256

    off = 0
    for nv in BLOCKS:
        pltpu.sync_copy(ekey_ref.at[pl.ds(base + off * 16, nv * 16)],
                        kbuf.at[pl.ds(0, nv * 16)])

        if level == 0:
            def per_vec(i, kv):
                bin12 = (kv >> np.uint32(20)).astype(_i32)
                plsc.addupdate_scatter(hist, [bin12 * 16 + iota], ones)
        else:
            def per_vec(i, kv):
                v = (kv >> np.uint32(shift)).astype(_i32)
                pos = jnp.zeros((16,), _i32)
                for s in (16, 8, 4, 2, 1):
                    p = plsc.load_gather(tbuf, [pos + (s - 1)])
                    pos = jnp.where(p <= v, pos + s, pos)
                slot = jnp.maximum(pos - 1, 0)
                pref = plsc.load_gather(tbuf, [slot])
                match = (pos > 0) & (pref == v)
                sub = ((kv >> np.uint32(subshift)) & submask).astype(_i32)
                idx = (slot * nsub + sub) * 16 + iota
                plsc.addupdate_scatter(hist, [idx], ones, mask=match)

        _sc_sweep([kbuf], nv, per_vec)
        off += nv

    _zero_ref(fold, outb)
    _fold_lanes(hist, fold, nbins)
    pltpu.sync_copy(fold, part_ref.at[wid])


def _make_sc_ehist(level):
    nbins = {0: 4096, 1: 4608, 2: 4608, 3: 288}[level]
    outb = {0: 4096, 1: 4608, 2: 4608, 3: 384}[level]
    histwords = {0: 65536, 1: 73728, 2: 73728, 3: 6144}[level]
    scratch = [pltpu.VMEM((8192,), _u32),
               pltpu.VMEM((histwords,), _i32),
               pltpu.VMEM((outb,), _i32)]
    if level > 0:
        scratch += [pltpu.VMEM((128,), _i32)]

    def body(*args):
        if level == 0:
            ekey_ref, part_ref, kbuf, hist, fold = args
            _sc_ehist_body(ekey_ref, part_ref, kbuf, hist, fold, level=0,
                           nbins=nbins, outb=outb, histwords=histwords)
        else:
            ekey_ref, tgt_ref, part_ref, kbuf, hist, fold, tbuf = args
            _sc_ehist_body(ekey_ref, part_ref, kbuf, hist, fold, level=level,
                           nbins=nbins, outb=outb, histwords=histwords,
                           tgt_ref=tgt_ref, tbuf=tbuf)

    return pl.kernel(
        body,
        out_type=jax.ShapeDtypeStruct((NT, outb), _i32),
        mesh=_MESH,
        compiler_params=pltpu.CompilerParams(needs_layout_passes=False),
        scratch_types=scratch,
    )


def _sc_shist_body(skey_ref, strat_ref, part_ref, kbuf, sbuf, hist, fold,
                   *, level, tgt_ref=None, tbuf=None):
    wid = _sc_wid()
    base = wid * (NVT * 16)
    _zero_ref(hist, 45056)
    iota = lax.broadcasted_iota(_i32, (16,), 0)
    ones = jnp.full((16,), 1, _i32)

    if level > 0:
        pltpu.sync_copy(tgt_ref, tbuf)
        shift = {1: 24, 2: 16, 3: 8}[level]
        subshift = {1: 16, 2: 8, 3: 0}[level]

    off = 0
    for nv in BLOCKS:
        pltpu.sync_copy(skey_ref.at[pl.ds(base + off * 16, nv * 16)],
                        kbuf.at[pl.ds(0, nv * 16)])
        pltpu.sync_copy(strat_ref.at[pl.ds(base + off * 16, nv * 16)],
                        sbuf.at[pl.ds(0, nv * 16)])

        if level == 0:
            def per_vec(i, kv, st):
                sub = (kv >> np.uint32(24)).astype(_i32)
                plsc.addupdate_scatter(hist, [(st * 256 + sub) * 16 + iota],
                                       ones)
        else:
            def per_vec(i, kv, st):
                pref = plsc.load_gather(tbuf, [st])
                match = (kv >> np.uint32(shift)).astype(_i32) == pref
                sub = ((kv >> np.uint32(subshift)) & np.uint32(0xFF)) \
                    .astype(_i32)
                idx = (st * 256 + sub) * 16 + iota
                plsc.addupdate_scatter(hist, [idx], ones, mask=match)

        _sc_sweep([kbuf, sbuf], nv, per_vec)
        off += nv

    _fold_lanes(hist, fold, 2560)
    pltpu.sync_copy(fold, part_ref.at[wid])


def _make_sc_shist(level):
    scratch = [pltpu.VMEM((8192,), _u32),
               pltpu.VMEM((8192,), _i32),
               pltpu.VMEM((45056,), _i32),
               pltpu.VMEM((2560,), _i32)]
    if level > 0:
        scratch += [pltpu.VMEM((128,), _i32)]

    def body(*args):
        if level == 0:
            skey_ref, strat_ref, part_ref, kbuf, sbuf, hist, fold = args
            _sc_shist_body(skey_ref, strat_ref, part_ref, kbuf, sbuf, hist,
                           fold, level=0)
        else:
            (skey_ref, strat_ref, tgt_ref, part_ref, kbuf, sbuf, hist, fold,
             tbuf) = args
            _sc_shist_body(skey_ref, strat_ref, part_ref, kbuf, sbuf, hist,
                           fold, level=level, tgt_ref=tgt_ref, tbuf=tbuf)

    return pl.kernel(
        body,
        out_type=jax.ShapeDtypeStruct((NT, 2560), _i32),
        mesh=_MESH,
        compiler_params=pltpu.CompilerParams(needs_layout_passes=False),
        scratch_types=scratch,
    )


_SC_EHIST = {lvl: _make_sc_ehist(lvl) for lvl in range(4)}
_SC_SHIST = {lvl: _make_sc_shist(lvl) for lvl in range(4)}


def kernel(entropy_map, node_scales, budget):
    e2 = jnp.pad(entropy_map, (0, PAD - N),
                 constant_values=np.float32(np.inf)).reshape(ROWS, 128)
    s2 = jnp.pad(node_scales.astype(_i32), (0, PAD - N)).reshape(ROWS, 128)

    ekey2, ppart = _tc_ekey(e2, s2)
    ekey1 = ekey2.reshape(PAD)

    eh0 = _SC_EHIST[0](ekey1).reshape(NT, 32, 128)
    etgt, escal = _tc_me0(eh0, ppart)
    eh1 = _SC_EHIST[1](ekey1, etgt.reshape(128)).reshape(NT, 36, 128)
    etgt = _tc_me12(eh1, etgt, 8)
    eh2 = _SC_EHIST[2](ekey1, etgt.reshape(128)).reshape(NT, 36, 128)
    etgt = _tc_me12(eh2, etgt, 8)
    eh3 = _SC_EHIST[3](ekey1, etgt.reshape(128)).reshape(NT, 3, 128)
    qb = _tc_me3(eh3, etgt)

    skey2, strat2 = _tc_skey(e2, s2, qb, escal)
    skey1 = skey2.reshape(PAD)
    strat1 = strat2.reshape(PAD)

    budget = jnp.asarray(budget, _i32)
    g = jnp.gcd(budget, N)
    bscal = jnp.zeros((1, 128), _i32)
    bscal = bscal.at[0, 0].set(budget // g).at[0, 1].set(N // g)

    sh0 = _SC_SHIST[0](skey1, strat1).reshape(NT, 20, 128)
    stgt = _tc_ms(sh0, bscal, bscal, level=0)
    sh1 = _SC_SHIST[1](skey1, strat1, stgt.reshape(128)).reshape(NT, 20, 128)
    stgt = _tc_ms(sh1, stgt, bscal, level=1)
    sh2 = _SC_SHIST[2](skey1, strat1, stgt.reshape(128)).reshape(NT, 20, 128)
    stgt = _tc_ms(sh2, stgt, bscal, level=2)
    sh3 = _SC_SHIST[3](skey1, strat1, stgt.reshape(128)).reshape(NT, 20, 128)
    tsel = _tc_ms3(sh3, stgt)

    probs2, sel2 = _tc_final(e2, s2, skey2, strat2, tsel, escal)
    probs = probs2.reshape(PAD)[:N]
    mask = sel2.reshape(PAD)[:N].astype(jnp.bool_)
    return probs, mask


# speedup vs baseline: 20.9383x; 1.1956x over previous
import functools

import numpy as np
import jax
import jax.numpy as jnp
from jax import lax
from jax.experimental import pallas as pl
from jax.experimental.pallas import tpu as pltpu
from jax.experimental.pallas import tpu_sc as plsc

N = 1_000_000
PAD = 1_000_448
ROWS = PAD // 128
NT = 32
NVT = 1954
BLOCKS = (512, 512, 512, 418)
NUM_STRATA = 10

_i32 = jnp.int32
_u32 = jnp.uint32
_f32 = jnp.float32


_QQ = np.array([0, 1203982323, 1212370931, 1217559543, 1220759539,
                1223959536, 1225948151, 1227548149, 1229148147, 1230748146,
                1232348144], dtype=np.uint32).view(np.float32)
_low9 = np.floor(_QQ).astype(np.int32)[1:10]
_high9 = np.ceil(_QQ).astype(np.int32)[1:10]
_hw9 = (_QQ - np.floor(_QQ)).astype(np.float32)[1:10]
_lw9 = (np.float32(1.0) - _hw9).astype(np.float32)
_RANKS18 = np.empty(18, np.int32)
_RANKS18[0::2] = _low9
_RANKS18[1::2] = _high9
_ranks_np = np.zeros((1, 128), np.int32)
_ranks_np[0, :18] = _RANKS18
_wts_np = np.zeros((1, 128), np.float32)
_wts_np[0, :9] = _lw9
_wts_np[0, 32:41] = _hw9

_CONSTS = {}


def _consts():
    if _CONSTS:
        return _CONSTS
    u = jax.random.uniform(jax.random.key(42), (N,), minval=1e-12, maxval=1.0)
    g = jnp.pad(-jnp.log(-jnp.log(u)), (0, PAD - N)).reshape(ROWS, 128)
    out = {"gumbel2d": g,
           "ranks_op": jnp.asarray(_ranks_np),
           "wts_op": jnp.asarray(_wts_np)}
    if not isinstance(g, jax.core.Tracer):
        _CONSTS.update(out)
    return out

_LOG1EM8 = np.float32(np.log(np.float32(1e-8)))
_SENT = 0x1FF


def _mono_u32(x):
    b = lax.bitcast_convert_type(x, _u32)
    neg = (b & np.uint32(0x80000000)) != 0
    return jnp.where(neg, ~b, b | np.uint32(0x80000000))


def _valid2d():
    r = lax.broadcasted_iota(_i32, (ROWS, 128), 0)
    c = lax.broadcasted_iota(_i32, (ROWS, 128), 1)
    return (r * 128 + c) < N


def _tc_ekey_body(e_ref, s_ref, ekey_ref, ppart_ref):
    e = e_ref[...]
    s = s_ref[...]
    ekey_ref[...] = _mono_u32(e)
    beta = np.float32(1.0) + np.float32(0.025) * s.astype(_f32)
    p = jnp.exp(beta * jnp.clip(e, -10.0, 10.0))
    p = jnp.where(_valid2d(), p, np.float32(0.0))
    ppart_ref[...] = jnp.sum(p, axis=0, keepdims=True)


def _tc_ekey(e2, s2):
    return pl.pallas_call(
        _tc_ekey_body,
        out_shape=[jax.ShapeDtypeStruct((ROWS, 128), _u32),
                   jax.ShapeDtypeStruct((1, 128), _f32)],
    )(e2, s2)


def _cumsum_lanes(x):
    col = lax.broadcasted_iota(_i32, x.shape, 1)
    for k in (1, 2, 4, 8, 16, 32, 64):
        sh = pltpu.roll(x, k, axis=1)
        x = x + jnp.where(col >= k, sh, jnp.zeros_like(x))
    return x


def _cumsum_rows(x):
    row = lax.broadcasted_iota(_i32, x.shape, 0)
    k = 1
    while k < x.shape[0]:
        sh = pltpu.roll(x, k, axis=0)
        x = x + jnp.where(row >= k, sh, jnp.zeros_like(x))
        k *= 2
    return x


_COL128 = None


def _col128():
    return lax.broadcasted_iota(_i32, (1, 128), 1)


def _get(vec, j):
    return jnp.sum(jnp.where(_col128() == j, vec, jnp.zeros_like(vec)))


def _put(acc, j, val):
    v = jnp.broadcast_to(jnp.asarray(val, acc.dtype), acc.shape)
    return jnp.where(_col128() == j, v, acc)


def _tc_me0_body(part_ref, ppart_ref, ranks_ref, etgt_ref, escal_ref):
    h3 = part_ref[...]
    h = jnp.sum(h3, axis=0)
    lane_cum = _cumsum_lanes(h)
    rowtot = lane_cum[:, 127:128]
    rowoff = _cumsum_rows(rowtot) - rowtot
    cum = lane_cum + rowoff
    binidx = (lax.broadcasted_iota(_i32, (32, 128), 0) * 128
              + lax.broadcasted_iota(_i32, (32, 128), 1))
    ranks = ranks_ref[...]
    etgt = jnp.zeros((1, 128), _i32)
    for j in range(18):
        r = _get(ranks, j)
        B = jnp.sum((cum <= r).astype(_i32))
        at = binidx == B
        hB = jnp.sum(jnp.where(at, h, 0))
        cB = jnp.sum(jnp.where(at, cum, 0))
        etgt = _put(etgt, j, B)
        etgt = _put(etgt, 32 + j, r - (cB - hB))
    col = _col128()
    etgt = jnp.where((col >= 18) & (col < 32),
                     jnp.full_like(etgt, np.int32(0x7FFFFFFF)), etgt)
    etgt_ref[...] = etgt
    S = jnp.sum(ppart_ref[...])
    fb = S <= np.float32(1e-8)
    escal = jnp.zeros((1, 128), _f32)
    escal = _put(escal, 0, jnp.where(fb, np.float32(0.0),
                                     np.float32(1.0) / S))
    escal = _put(escal, 1, jnp.where(fb, np.float32(1.0) / np.float32(N),
                                     np.float32(0.0)))
    escal = _put(escal, 2, jnp.where(fb, np.float32(0.0),
                                     np.float32(1.0)))
    escal = _put(escal, 3, jnp.where(fb,
                                     jnp.log(np.float32(1.0) / np.float32(N)),
                                     -jnp.log(S)))
    escal_ref[...] = escal


def _tc_me0(part, ppart):
    return pl.pallas_call(
        _tc_me0_body,
        out_shape=[jax.ShapeDtypeStruct((1, 128), _i32),
                   jax.ShapeDtypeStruct((1, 128), _f32)],
    )(part, ppart, _consts()["ranks_op"])


def _seg_cum_256(h):
    lane_cum = _cumsum_lanes(h)
    rowtot = lane_cum[:, 127:128]
    prev = pltpu.roll(rowtot, 1, axis=0)
    row = lax.broadcasted_iota(_i32, lane_cum.shape, 0)
    odd = (row % 2) == 1
    return lane_cum + jnp.where(odd, prev, jnp.zeros_like(prev))


def _owner18(prev, j):
    pj = _get(prev, j)
    owner = jnp.asarray(j, _i32)
    for t2 in range(18):
        owner = jnp.where(_get(prev, t2) == pj, jnp.asarray(t2, _i32), owner)
    return owner


def _tc_me12_body(part_ref, prev_ref, etgt_ref, *, shift_out):
    h3 = part_ref[...]
    h = jnp.sum(h3, axis=0)
    cum = _seg_cum_256(h)
    row = lax.broadcasted_iota(_i32, (36, 128), 0)
    sub = ((row % 2) * 128 + lax.broadcasted_iota(_i32, (36, 128), 1))
    tgt = row // 2
    prev = prev_ref[...]
    etgt = jnp.zeros((1, 128), _i32)
    for j in range(18):
        r = _get(prev, 32 + j)
        pref = _get(prev, j)
        mine = tgt == _owner18(prev, j)
        B = jnp.sum((mine & (cum <= r)).astype(_i32))
        at = mine & (sub == B)
        hB = jnp.sum(jnp.where(at, h, 0))
        cB = jnp.sum(jnp.where(at, cum, 0))
        etgt = _put(etgt, j, (pref << shift_out) | B)
        etgt = _put(etgt, 32 + j, r - (cB - hB))
    col = _col128()
    etgt = jnp.where((col >= 18) & (col < 32),
                     jnp.full_like(etgt, np.int32(0x7FFFFFFF)), etgt)
    etgt_ref[...] = etgt


def _tc_me12(part, prev, shift_out):
    return pl.pallas_call(
        functools.partial(_tc_me12_body, shift_out=shift_out),
        out_shape=jax.ShapeDtypeStruct((1, 128), _i32),
    )(part, prev)


def _tc_me3_body(part_ref, prev_ref, wts_ref, qb_ref):
    h3 = part_ref[...]
    h = jnp.sum(h3, axis=0)
    col = lax.broadcasted_iota(_i32, (3, 128), 1)
    cum = h
    for k in (1, 2, 4, 8):
        sh = pltpu.roll(cum, k, axis=1)
        cum = cum + jnp.where((col % 16) >= k, sh, jnp.zeros_like(sh))
    row = lax.broadcasted_iota(_i32, (3, 128), 0)
    flat = row * 128 + col
    tgt = flat // 16
    prev = prev_ref[...]
    wts = wts_ref[...]
    vals = []
    for j in range(18):
        r = _get(prev, 32 + j)
        pref = _get(prev, j)
        mine = tgt == _owner18(prev, j)
        B0 = jnp.sum((mine & (cum <= r)).astype(_i32))
        key = ((pref.astype(_u32) << np.uint32(4)) | B0.astype(_u32))
        bits = jnp.where((key & np.uint32(0x80000000)) != 0,
                         key ^ np.uint32(0x80000000), ~key)
        vals.append(lax.bitcast_convert_type(bits, _f32))
    qb = jnp.zeros((1, 128), _f32)
    for k in range(9):
        qb = _put(qb, k, vals[2 * k] * _get(wts, k)
                  + vals[2 * k + 1] * _get(wts, 32 + k))
    qb_ref[...] = qb


def _tc_me3(part, prev):
    return pl.pallas_call(
        _tc_me3_body,
        out_shape=jax.ShapeDtypeStruct((1, 128), _f32),
    )(part, prev, _consts()["wts_op"])


def _tc_skey_body(e_ref, s_ref, g_ref, qb_ref, escal_ref, skey_ref, strat_ref):
    e = e_ref[...]
    s = s_ref[...]
    escal = escal_ref[...]
    qb = qb_ref[...]
    A = _get(escal, 2)
    C = _get(escal, 3)
    beta = np.float32(1.0) + np.float32(0.025) * s.astype(_f32)
    be = beta * jnp.clip(e, -10.0, 10.0)
    score = jnp.maximum(A * be + C, _LOG1EM8) + g_ref[...]
    skey_ref[...] = _mono_u32(score)
    strat = jnp.zeros_like(s)
    for k in range(9):
        strat = strat + (_get(qb, k) <= e).astype(_i32)
    strat_ref[...] = jnp.where(_valid2d(), strat, NUM_STRATA)


def _tc_skey(e2, s2, qb, escal):
    return pl.pallas_call(
        _tc_skey_body,
        out_shape=[jax.ShapeDtypeStruct((ROWS, 128), _u32),
                   jax.ShapeDtypeStruct((ROWS, 128), _i32)],
    )(e2, s2, _consts()["gumbel2d"], qb, escal)


def _tc_ms_body(part_ref, prev_ref, bscal_ref, stgt_ref, *, level):
    h3 = part_ref[...]
    h = jnp.sum(h3, axis=0)
    cum = _seg_cum_256(h)
    row = lax.broadcasted_iota(_i32, (20, 128), 0)
    sub = ((row % 2) * 128 + lax.broadcasted_iota(_i32, (20, 128), 1))
    strat = row // 2
    prev = prev_ref[...]
    bscal = bscal_ref[...]
    stgt = jnp.zeros((1, 128), _i32)
    for i in range(NUM_STRATA):
        mine = strat == i
        tot = jnp.sum(jnp.where(mine, h, 0))
        if level == 0:
            q1 = _get(bscal, 0)
            q2 = _get(bscal, 1)
            b = jnp.minimum(jnp.maximum(1, (tot * q1) // q2), tot)
            need = b
            pref = jnp.asarray(0, _i32)
        else:
            need = _get(prev, 16 + i)
            pref = _get(prev, i)
            b = _get(prev, 32 + i)
        suf = tot - cum + h
        Bv = jnp.sum((mine & (suf >= need)).astype(_i32)) - 1
        at = mine & (sub == Bv)
        hB = jnp.sum(jnp.where(at, h, 0))
        sB = jnp.sum(jnp.where(at, suf, 0))
        dead = b <= 0
        stgt = _put(stgt, i, jnp.where(dead, _SENT << (8 * level),
                                       (pref << 8) | jnp.maximum(Bv, 0)))
        stgt = _put(stgt, 16 + i, jnp.where(dead, 0, need - (sB - hB)))
        stgt = _put(stgt, 32 + i, b)
    stgt = _put(stgt, NUM_STRATA, _SENT << (8 * level))
    stgt_ref[...] = stgt


def _tc_ms(part, prev, bscal, level):
    return pl.pallas_call(
        functools.partial(_tc_ms_body, level=level),
        out_shape=jax.ShapeDtypeStruct((1, 128), _i32),
    )(part, prev, bscal)


def _tc_ms3_body(part_ref, prev_ref, tsel_ref):
    h3 = part_ref[...]
    h = jnp.sum(h3, axis=0)
    cum = _seg_cum_256(h)
    row = lax.broadcasted_iota(_i32, (20, 128), 0)
    strat = row // 2
    prev = prev_ref[...]
    tsel = jnp.full((1, 128), np.uint32(0xFFFFFFFF), _u32)
    for i in range(NUM_STRATA):
        mine = strat == i
        need = _get(prev, 16 + i)
        pref = _get(prev, i)
        b = _get(prev, 32 + i)
        tot = jnp.sum(jnp.where(mine, h, 0))
        suf = tot - cum + h
        Bv = jnp.sum((mine & (suf >= need)).astype(_i32)) - 1
        T = ((pref.astype(_u32) << np.uint32(8))
             | jnp.maximum(Bv, 0).astype(_u32))
        tsel = _put(tsel, i, jnp.where(b <= 0, np.uint32(0xFFFFFFFF), T))
    tsel_ref[...] = tsel


def _tc_ms3(part, prev):
    return pl.pallas_call(
        _tc_ms3_body,
        out_shape=jax.ShapeDtypeStruct((1, 128), _u32),
    )(part, prev)


def _tc_final_body(e_ref, s_ref, skey_ref, strat_ref, tsel_ref, escal_ref,
                   probs_ref, sel_ref):
    e = e_ref[...]
    s = s_ref[...]
    escal = escal_ref[...]
    tsel = tsel_ref[...]
    beta = np.float32(1.0) + np.float32(0.025) * s.astype(_f32)
    p = jnp.exp(beta * jnp.clip(e, -10.0, 10.0))
    probs_ref[...] = p * _get(escal, 0) + _get(escal, 1)
    strat = strat_ref[...]
    tsel_i = lax.bitcast_convert_type(tsel, _i32)
    T = jnp.full(strat.shape, np.uint32(0xFFFFFFFF), _u32)
    for i in range(NUM_STRATA):
        Ti = lax.bitcast_convert_type(_get(tsel_i, i), _u32)
        T = jnp.where(strat == i, Ti, T)
    sel_ref[...] = (skey_ref[...] >= T).astype(_i32)


def _tc_final(e2, s2, skey2, strat2, tsel, escal):
    return pl.pallas_call(
        _tc_final_body,
        out_shape=[jax.ShapeDtypeStruct((ROWS, 128), _f32),
                   jax.ShapeDtypeStruct((ROWS, 128), _i32)],
    )(e2, s2, skey2, strat2, tsel, escal)


_MESH = plsc.VectorSubcoreMesh(core_axis_name="c", subcore_axis_name="s",
                               num_cores=2, num_subcores=16)


def _zero_ref(ref, nwords):
    z = jnp.zeros((16,), _i32)
    assert nwords % 128 == 0

    def body(j, carry):
        for u in range(8):
            ref[pl.ds(j * 128 + u * 16, 16)] = z
        return carry

    lax.fori_loop(0, nwords // 128, body, 0)


def _fold_lanes(hist, fold, nbins):
    iota = lax.broadcasted_iota(_i32, (16,), 0)
    assert nbins % 32 == 0

    def body(j, carry):
        for u in range(2):
            bins = iota + (j * 32 + u * 16)
            acc = jnp.zeros((16,), _i32)
            for L in range(16):
                acc = acc + plsc.load_gather(hist, [bins * 16 + L])
            fold[pl.ds(j * 32 + u * 16, 16)] = acc
        return carry

    lax.fori_loop(0, nbins // 32, body, 0)


def _sc_wid():
    return lax.axis_index("s") * 2 + lax.axis_index("c")


def _sc_sweep(kbuf_list, nv, per_vec):
    n4 = nv // 4

    def body(i, carry):
        for u in range(4):
            j = i * 4 + u
            vecs = [b[pl.ds(j * 16, 16)] for b in kbuf_list]
            per_vec(j, *vecs)
        return carry

    lax.fori_loop(0, n4, body, 0)
    for j in range(n4 * 4, nv):
        vecs = [b[pl.ds(j * 16, 16)] for b in kbuf_list]
        per_vec(j, *vecs)


def _sc_ehist_body(ekey_ref, part_ref, kbuf, hist, fold, *, level, nbins,
                   outb, histwords, tgt_ref=None, tbuf=None):
    wid = _sc_wid()
    base = wid * (NVT * 16)
    _zero_ref(hist, histwords)
    iota = lax.broadcasted_iota(_i32, (16,), 0)
    ones = jnp.full((16,), 1, _i32)

    if level > 0:
        pltpu.sync_copy(tgt_ref, tbuf)
        t0 = tbuf[pl.ds(0, 16)]
        t1 = tbuf[pl.ds(16, 16)]
        prefs = [t0[t] for t in range(16)] + [t1[0], t1[1]]
        shift = {1: 20, 2: 12, 3: 4}[level]
        submask = np.uint32(0xF if level == 3 else 0xFF)
        subshift = {1: 12, 2: 4, 3: 0}[level]
        nsub = 16 if level == 3 else 256

    off = 0
    for nv in BLOCKS:
        pltpu.sync_copy(ekey_ref.at[pl.ds(base + off * 16, nv * 16)],
                        kbuf.at[pl.ds(0, nv * 16)])

        if level == 0:
            def per_vec(i, kv):
                bin12 = (kv >> np.uint32(20)).astype(_i32)
                plsc.addupdate_scatter(hist, [bin12 * 16 + iota], ones)
        else:
            def per_vec(i, kv):
                v = (kv >> np.uint32(shift)).astype(_i32)
                pos = jnp.zeros((16,), _i32)
                for t in range(18):
                    pos = pos + (prefs[t] <= v).astype(_i32)
                slot = jnp.maximum(pos - 1, 0)
                pref = plsc.load_gather(tbuf, [slot])
                match = (pos > 0) & (pref == v)
                sub = ((kv >> np.uint32(subshift)) & submask).astype(_i32)
                idx = (slot * nsub + sub) * 16 + iota
                plsc.addupdate_scatter(hist, [idx], ones, mask=match)

        _sc_sweep([kbuf], nv, per_vec)
        off += nv

    _zero_ref(fold, outb)
    _fold_lanes(hist, fold, nbins)
    pltpu.sync_copy(fold, part_ref.at[wid])


def _make_sc_ehist(level):
    nbins = {0: 4096, 1: 4608, 2: 4608, 3: 288}[level]
    outb = {0: 4096, 1: 4608, 2: 4608, 3: 384}[level]
    histwords = {0: 65536, 1: 73728, 2: 73728, 3: 6144}[level]
    scratch = [pltpu.VMEM((8192,), _u32),
               pltpu.VMEM((histwords,), _i32),
               pltpu.VMEM((outb,), _i32)]
    if level > 0:
        scratch += [pltpu.VMEM((128,), _i32)]

    def body(*args):
        if level == 0:
            ekey_ref, part_ref, kbuf, hist, fold = args
            _sc_ehist_body(ekey_ref, part_ref, kbuf, hist, fold, level=0,
                           nbins=nbins, outb=outb, histwords=histwords)
        else:
            ekey_ref, tgt_ref, part_ref, kbuf, hist, fold, tbuf = args
            _sc_ehist_body(ekey_ref, part_ref, kbuf, hist, fold, level=level,
                           nbins=nbins, outb=outb, histwords=histwords,
                           tgt_ref=tgt_ref, tbuf=tbuf)

    return pl.kernel(
        body,
        out_type=jax.ShapeDtypeStruct((NT, outb), _i32),
        mesh=_MESH,
        compiler_params=pltpu.CompilerParams(needs_layout_passes=False),
        scratch_types=scratch,
    )


def _sc_shist_body(skey_ref, strat_ref, part_ref, kbuf, sbuf, hist, fold,
                   *, level, tgt_ref=None, tbuf=None):
    wid = _sc_wid()
    base = wid * (NVT * 16)
    _zero_ref(hist, 45056)
    iota = lax.broadcasted_iota(_i32, (16,), 0)
    ones = jnp.full((16,), 1, _i32)

    if level > 0:
        pltpu.sync_copy(tgt_ref, tbuf)
        shift = {1: 24, 2: 16, 3: 8}[level]
        subshift = {1: 16, 2: 8, 3: 0}[level]

    off = 0
    for nv in BLOCKS:
        pltpu.sync_copy(skey_ref.at[pl.ds(base + off * 16, nv * 16)],
                        kbuf.at[pl.ds(0, nv * 16)])
        pltpu.sync_copy(strat_ref.at[pl.ds(base + off * 16, nv * 16)],
                        sbuf.at[pl.ds(0, nv * 16)])

        if level == 0:
            def per_vec(i, kv, st):
                sub = (kv >> np.uint32(24)).astype(_i32)
                plsc.addupdate_scatter(hist, [(st * 256 + sub) * 16 + iota],
                                       ones)
        else:
            def per_vec(i, kv, st):
                pref = plsc.load_gather(tbuf, [st])
                match = (kv >> np.uint32(shift)).astype(_i32) == pref
                sub = ((kv >> np.uint32(subshift)) & np.uint32(0xFF)) \
                    .astype(_i32)
                idx = (st * 256 + sub) * 16 + iota
                plsc.addupdate_scatter(hist, [idx], ones, mask=match)

        _sc_sweep([kbuf, sbuf], nv, per_vec)
        off += nv

    _fold_lanes(hist, fold, 2560)
    pltpu.sync_copy(fold, part_ref.at[wid])


def _make_sc_shist(level):
    scratch = [pltpu.VMEM((8192,), _u32),
               pltpu.VMEM((8192,), _i32),
               pltpu.VMEM((45056,), _i32),
               pltpu.VMEM((2560,), _i32)]
    if level > 0:
        scratch += [pltpu.VMEM((128,), _i32)]

    def body(*args):
        if level == 0:
            skey_ref, strat_ref, part_ref, kbuf, sbuf, hist, fold = args
            _sc_shist_body(skey_ref, strat_ref, part_ref, kbuf, sbuf, hist,
                           fold, level=0)
        else:
            (skey_ref, strat_ref, tgt_ref, part_ref, kbuf, sbuf, hist, fold,
             tbuf) = args
            _sc_shist_body(skey_ref, strat_ref, part_ref, kbuf, sbuf, hist,
                           fold, level=level, tgt_ref=tgt_ref, tbuf=tbuf)

    return pl.kernel(
        body,
        out_type=jax.ShapeDtypeStruct((NT, 2560), _i32),
        mesh=_MESH,
        compiler_params=pltpu.CompilerParams(needs_layout_passes=False),
        scratch_types=scratch,
    )


_SC_EHIST = {lvl: _make_sc_ehist(lvl) for lvl in range(4)}
_SC_SHIST = {lvl: _make_sc_shist(lvl) for lvl in range(4)}


def kernel(entropy_map, node_scales, budget):
    e2 = jnp.pad(entropy_map, (0, PAD - N),
                 constant_values=np.float32(np.inf)).reshape(ROWS, 128)
    s2 = jnp.pad(node_scales.astype(_i32), (0, PAD - N)).reshape(ROWS, 128)

    ekey2, ppart = _tc_ekey(e2, s2)
    ekey1 = ekey2.reshape(PAD)

    eh0 = _SC_EHIST[0](ekey1).reshape(NT, 32, 128)
    etgt, escal = _tc_me0(eh0, ppart)
    eh1 = _SC_EHIST[1](ekey1, etgt.reshape(128)).reshape(NT, 36, 128)
    etgt = _tc_me12(eh1, etgt, 8)
    eh2 = _SC_EHIST[2](ekey1, etgt.reshape(128)).reshape(NT, 36, 128)
    etgt = _tc_me12(eh2, etgt, 8)
    eh3 = _SC_EHIST[3](ekey1, etgt.reshape(128)).reshape(NT, 3, 128)
    qb = _tc_me3(eh3, etgt)

    skey2, strat2 = _tc_skey(e2, s2, qb, escal)
    skey1 = skey2.reshape(PAD)
    strat1 = strat2.reshape(PAD)

    budget = jnp.asarray(budget, _i32)
    g = jnp.gcd(budget, N)
    bscal = jnp.zeros((1, 128), _i32)
    bscal = bscal.at[0, 0].set(budget // g).at[0, 1].set(N // g)

    sh0 = _SC_SHIST[0](skey1, strat1).reshape(NT, 20, 128)
    stgt = _tc_ms(sh0, bscal, bscal, level=0)
    sh1 = _SC_SHIST[1](skey1, strat1, stgt.reshape(128)).reshape(NT, 20, 128)
    stgt = _tc_ms(sh1, stgt, bscal, level=1)
    sh2 = _SC_SHIST[2](skey1, strat1, stgt.reshape(128)).reshape(NT, 20, 128)
    stgt = _tc_ms(sh2, stgt, bscal, level=2)
    sh3 = _SC_SHIST[3](skey1, strat1, stgt.reshape(128)).reshape(NT, 20, 128)
    tsel = _tc_ms3(sh3, stgt)

    probs2, sel2 = _tc_final(e2, s2, skey2, strat2, tsel, escal)
    probs = probs2.reshape(PAD)[:N]
    mask = sel2.reshape(PAD)[:N].astype(jnp.bool_)
    return probs, mask


# speedup vs baseline: 20.9806x; 1.0020x over previous
import functools

import numpy as np
import jax
import jax.numpy as jnp
from jax import lax
from jax.experimental import pallas as pl
from jax.experimental.pallas import tpu as pltpu
from jax.experimental.pallas import tpu_sc as plsc

N = 1_000_000
PAD = 1_000_448
ROWS = PAD // 128
NT = 32
NVT = 1954
BLOCKS = (512, 512, 512, 418)
NUM_STRATA = 10

_i32 = jnp.int32
_u32 = jnp.uint32
_f32 = jnp.float32


_QQ = np.array([0, 1203982323, 1212370931, 1217559543, 1220759539,
                1223959536, 1225948151, 1227548149, 1229148147, 1230748146,
                1232348144], dtype=np.uint32).view(np.float32)
_low9 = np.floor(_QQ).astype(np.int32)[1:10]
_high9 = np.ceil(_QQ).astype(np.int32)[1:10]
_hw9 = (_QQ - np.floor(_QQ)).astype(np.float32)[1:10]
_lw9 = (np.float32(1.0) - _hw9).astype(np.float32)
_RANKS18 = np.empty(18, np.int32)
_RANKS18[0::2] = _low9
_RANKS18[1::2] = _high9
_ranks_np = np.zeros((1, 128), np.int32)
_ranks_np[0, :18] = _RANKS18
_wts_np = np.zeros((1, 128), np.float32)
_wts_np[0, :9] = _lw9
_wts_np[0, 32:41] = _hw9

_CONSTS = {}


def _consts():
    if _CONSTS:
        return _CONSTS
    u = jax.random.uniform(jax.random.key(42), (N,), minval=1e-12, maxval=1.0)
    g = jnp.pad(-jnp.log(-jnp.log(u)), (0, PAD - N)).reshape(ROWS, 128)
    out = {"gumbel2d": g,
           "ranks_op": jnp.asarray(_ranks_np),
           "wts_op": jnp.asarray(_wts_np)}
    if not isinstance(g, jax.core.Tracer):
        _CONSTS.update(out)
    return out

_LOG1EM8 = np.float32(np.log(np.float32(1e-8)))
_SENT = 0x1FF


def _mono_u32(x):
    b = lax.bitcast_convert_type(x, _u32)
    neg = (b & np.uint32(0x80000000)) != 0
    return jnp.where(neg, ~b, b | np.uint32(0x80000000))


def _valid2d():
    r = lax.broadcasted_iota(_i32, (ROWS, 128), 0)
    c = lax.broadcasted_iota(_i32, (ROWS, 128), 1)
    return (r * 128 + c) < N


def _tc_ekey_body(e_ref, s_ref, ekey_ref, ppart_ref):
    e = e_ref[...]
    s = s_ref[...]
    ekey_ref[...] = _mono_u32(e)
    beta = np.float32(1.0) + np.float32(0.025) * s.astype(_f32)
    p = jnp.exp(beta * jnp.clip(e, -10.0, 10.0))
    p = jnp.where(_valid2d(), p, np.float32(0.0))
    ppart_ref[...] = jnp.sum(p, axis=0, keepdims=True)


def _tc_ekey(e2, s2):
    return pl.pallas_call(
        _tc_ekey_body,
        out_shape=[jax.ShapeDtypeStruct((ROWS, 128), _u32),
                   jax.ShapeDtypeStruct((1, 128), _f32)],
    )(e2, s2)


def _cumsum_lanes(x):
    col = lax.broadcasted_iota(_i32, x.shape, 1)
    for k in (1, 2, 4, 8, 16, 32, 64):
        sh = pltpu.roll(x, k, axis=1)
        x = x + jnp.where(col >= k, sh, jnp.zeros_like(x))
    return x


def _cumsum_rows(x):
    row = lax.broadcasted_iota(_i32, x.shape, 0)
    k = 1
    while k < x.shape[0]:
        sh = pltpu.roll(x, k, axis=0)
        x = x + jnp.where(row >= k, sh, jnp.zeros_like(x))
        k *= 2
    return x


_COL128 = None


def _col128():
    return lax.broadcasted_iota(_i32, (1, 128), 1)


def _get(vec, j):
    return jnp.sum(jnp.where(_col128() == j, vec, jnp.zeros_like(vec)))


def _put(acc, j, val):
    v = jnp.broadcast_to(jnp.asarray(val, acc.dtype), acc.shape)
    return jnp.where(_col128() == j, v, acc)


def _tc_me0_body(part_ref, ppart_ref, ranks_ref, etgt_ref, escal_ref):
    h3 = part_ref[...]
    h = jnp.sum(h3, axis=0)
    lane_cum = _cumsum_lanes(h)
    rowtot = lane_cum[:, 127:128]
    rowoff = _cumsum_rows(rowtot) - rowtot
    cum = lane_cum + rowoff
    binidx = (lax.broadcasted_iota(_i32, (32, 128), 0) * 128
              + lax.broadcasted_iota(_i32, (32, 128), 1))
    ranks = ranks_ref[...]
    etgt = jnp.zeros((1, 128), _i32)
    for j in range(18):
        r = _get(ranks, j)
        B = jnp.sum((cum <= r).astype(_i32))
        at = binidx == B
        hB = jnp.sum(jnp.where(at, h, 0))
        cB = jnp.sum(jnp.where(at, cum, 0))
        etgt = _put(etgt, j, B)
        etgt = _put(etgt, 32 + j, r - (cB - hB))
    col = _col128()
    etgt = jnp.where((col >= 18) & (col < 32),
                     jnp.full_like(etgt, np.int32(0x7FFFFFFF)), etgt)
    etgt_ref[...] = etgt
    S = jnp.sum(ppart_ref[...])
    fb = S <= np.float32(1e-8)
    escal = jnp.zeros((1, 128), _f32)
    escal = _put(escal, 0, jnp.where(fb, np.float32(0.0),
                                     np.float32(1.0) / S))
    escal = _put(escal, 1, jnp.where(fb, np.float32(1.0) / np.float32(N),
                                     np.float32(0.0)))
    escal = _put(escal, 2, jnp.where(fb, np.float32(0.0),
                                     np.float32(1.0)))
    escal = _put(escal, 3, jnp.where(fb,
                                     jnp.log(np.float32(1.0) / np.float32(N)),
                                     -jnp.log(S)))
    escal_ref[...] = escal


def _tc_me0(part, ppart):
    return pl.pallas_call(
        _tc_me0_body,
        out_shape=[jax.ShapeDtypeStruct((1, 128), _i32),
                   jax.ShapeDtypeStruct((1, 128), _f32)],
    )(part, ppart, _consts()["ranks_op"])


def _seg_cum_256(h):
    lane_cum = _cumsum_lanes(h)
    rowtot = lane_cum[:, 127:128]
    prev = pltpu.roll(rowtot, 1, axis=0)
    row = lax.broadcasted_iota(_i32, lane_cum.shape, 0)
    odd = (row % 2) == 1
    return lane_cum + jnp.where(odd, prev, jnp.zeros_like(prev))


def _owner18(prev, j):
    pj = _get(prev, j)
    owner = jnp.asarray(j, _i32)
    for t2 in range(18):
        owner = jnp.where(_get(prev, t2) == pj, jnp.asarray(t2, _i32), owner)
    return owner


def _tc_me12_body(part_ref, prev_ref, etgt_ref, *, shift_out):
    h3 = part_ref[...]
    h = jnp.sum(h3, axis=0)
    cum = _seg_cum_256(h)
    row = lax.broadcasted_iota(_i32, (36, 128), 0)
    sub = ((row % 2) * 128 + lax.broadcasted_iota(_i32, (36, 128), 1))
    tgt = row // 2
    prev = prev_ref[...]
    etgt = jnp.zeros((1, 128), _i32)
    for j in range(18):
        r = _get(prev, 32 + j)
        pref = _get(prev, j)
        mine = tgt == _owner18(prev, j)
        B = jnp.sum((mine & (cum <= r)).astype(_i32))
        at = mine & (sub == B)
        hB = jnp.sum(jnp.where(at, h, 0))
        cB = jnp.sum(jnp.where(at, cum, 0))
        etgt = _put(etgt, j, (pref << shift_out) | B)
        etgt = _put(etgt, 32 + j, r - (cB - hB))
    col = _col128()
    etgt = jnp.where((col >= 18) & (col < 32),
                     jnp.full_like(etgt, np.int32(0x7FFFFFFF)), etgt)
    etgt_ref[...] = etgt


def _tc_me12(part, prev, shift_out):
    return pl.pallas_call(
        functools.partial(_tc_me12_body, shift_out=shift_out),
        out_shape=jax.ShapeDtypeStruct((1, 128), _i32),
    )(part, prev)


def _tc_me3_body(part_ref, prev_ref, wts_ref, qb_ref):
    h3 = part_ref[...]
    h = jnp.sum(h3, axis=0)
    col = lax.broadcasted_iota(_i32, (3, 128), 1)
    cum = h
    for k in (1, 2, 4, 8):
        sh = pltpu.roll(cum, k, axis=1)
        cum = cum + jnp.where((col % 16) >= k, sh, jnp.zeros_like(sh))
    row = lax.broadcasted_iota(_i32, (3, 128), 0)
    flat = row * 128 + col
    tgt = flat // 16
    prev = prev_ref[...]
    wts = wts_ref[...]
    vals = []
    for j in range(18):
        r = _get(prev, 32 + j)
        pref = _get(prev, j)
        mine = tgt == _owner18(prev, j)
        B0 = jnp.sum((mine & (cum <= r)).astype(_i32))
        key = ((pref.astype(_u32) << np.uint32(4)) | B0.astype(_u32))
        bits = jnp.where((key & np.uint32(0x80000000)) != 0,
                         key ^ np.uint32(0x80000000), ~key)
        vals.append(lax.bitcast_convert_type(bits, _f32))
    qb = jnp.zeros((1, 128), _f32)
    for k in range(9):
        qb = _put(qb, k, vals[2 * k] * _get(wts, k)
                  + vals[2 * k + 1] * _get(wts, 32 + k))
    qb_ref[...] = qb


def _tc_me3(part, prev):
    return pl.pallas_call(
        _tc_me3_body,
        out_shape=jax.ShapeDtypeStruct((1, 128), _f32),
    )(part, prev, _consts()["wts_op"])


def _tc_skey_body(e_ref, s_ref, g_ref, qb_ref, escal_ref, skey_ref, strat_ref):
    e = e_ref[...]
    s = s_ref[...]
    escal = escal_ref[...]
    qb = qb_ref[...]
    A = _get(escal, 2)
    C = _get(escal, 3)
    beta = np.float32(1.0) + np.float32(0.025) * s.astype(_f32)
    be = beta * jnp.clip(e, -10.0, 10.0)
    score = jnp.maximum(A * be + C, _LOG1EM8) + g_ref[...]
    skey_ref[...] = _mono_u32(score)
    strat = jnp.zeros_like(s)
    for k in range(9):
        strat = strat + (_get(qb, k) <= e).astype(_i32)
    strat_ref[...] = jnp.where(_valid2d(), strat, NUM_STRATA)


def _tc_skey(e2, s2, qb, escal):
    return pl.pallas_call(
        _tc_skey_body,
        out_shape=[jax.ShapeDtypeStruct((ROWS, 128), _u32),
                   jax.ShapeDtypeStruct((ROWS, 128), _i32)],
    )(e2, s2, _consts()["gumbel2d"], qb, escal)


def _tc_ms_body(part_ref, prev_ref, bscal_ref, stgt_ref, *, level):
    h3 = part_ref[...]
    h = jnp.sum(h3, axis=0)
    cum = _seg_cum_256(h)
    row = lax.broadcasted_iota(_i32, (20, 128), 0)
    sub = ((row % 2) * 128 + lax.broadcasted_iota(_i32, (20, 128), 1))
    strat = row // 2
    prev = prev_ref[...]
    bscal = bscal_ref[...]
    stgt = jnp.zeros((1, 128), _i32)
    for i in range(NUM_STRATA):
        mine = strat == i
        tot = jnp.sum(jnp.where(mine, h, 0))
        if level == 0:
            q1 = _get(bscal, 0)
            q2 = _get(bscal, 1)
            b = jnp.minimum(jnp.maximum(1, (tot * q1) // q2), tot)
            need = b
            pref = jnp.asarray(0, _i32)
        else:
            need = _get(prev, 16 + i)
            pref = _get(prev, i)
            b = _get(prev, 32 + i)
        suf = tot - cum + h
        Bv = jnp.sum((mine & (suf >= need)).astype(_i32)) - 1
        at = mine & (sub == Bv)
        hB = jnp.sum(jnp.where(at, h, 0))
        sB = jnp.sum(jnp.where(at, suf, 0))
        dead = b <= 0
        stgt = _put(stgt, i, jnp.where(dead, _SENT << (8 * level),
                                       (pref << 8) | jnp.maximum(Bv, 0)))
        stgt = _put(stgt, 16 + i, jnp.where(dead, 0, need - (sB - hB)))
        stgt = _put(stgt, 32 + i, b)
    stgt = _put(stgt, NUM_STRATA, _SENT << (8 * level))
    stgt_ref[...] = stgt


def _tc_ms(part, prev, bscal, level):
    return pl.pallas_call(
        functools.partial(_tc_ms_body, level=level),
        out_shape=jax.ShapeDtypeStruct((1, 128), _i32),
    )(part, prev, bscal)


def _tc_ms3_body(part_ref, prev_ref, tsel_ref):
    h3 = part_ref[...]
    h = jnp.sum(h3, axis=0)
    cum = _seg_cum_256(h)
    row = lax.broadcasted_iota(_i32, (20, 128), 0)
    strat = row // 2
    prev = prev_ref[...]
    tsel = jnp.full((1, 128), np.uint32(0xFFFFFFFF), _u32)
    for i in range(NUM_STRATA):
        mine = strat == i
        need = _get(prev, 16 + i)
        pref = _get(prev, i)
        b = _get(prev, 32 + i)
        tot = jnp.sum(jnp.where(mine, h, 0))
        suf = tot - cum + h
        Bv = jnp.sum((mine & (suf >= need)).astype(_i32)) - 1
        T = ((pref.astype(_u32) << np.uint32(8))
             | jnp.maximum(Bv, 0).astype(_u32))
        tsel = _put(tsel, i, jnp.where(b <= 0, np.uint32(0xFFFFFFFF), T))
    tsel_ref[...] = tsel


def _tc_ms3(part, prev):
    return pl.pallas_call(
        _tc_ms3_body,
        out_shape=jax.ShapeDtypeStruct((1, 128), _u32),
    )(part, prev)


def _tc_final_body(e_ref, s_ref, skey_ref, strat_ref, tsel_ref, escal_ref,
                   probs_ref, sel_ref):
    e = e_ref[...]
    s = s_ref[...]
    escal = escal_ref[...]
    tsel = tsel_ref[...]
    beta = np.float32(1.0) + np.float32(0.025) * s.astype(_f32)
    p = jnp.exp(beta * jnp.clip(e, -10.0, 10.0))
    probs_ref[...] = p * _get(escal, 0) + _get(escal, 1)
    strat = strat_ref[...]
    tsel_i = lax.bitcast_convert_type(tsel, _i32)
    T = jnp.full(strat.shape, np.uint32(0xFFFFFFFF), _u32)
    for i in range(NUM_STRATA):
        Ti = lax.bitcast_convert_type(_get(tsel_i, i), _u32)
        T = jnp.where(strat == i, Ti, T)
    sel_ref[...] = (skey_ref[...] >= T).astype(_i32)


def _tc_final(e2, s2, skey2, strat2, tsel, escal):
    return pl.pallas_call(
        _tc_final_body,
        out_shape=[jax.ShapeDtypeStruct((ROWS, 128), _f32),
                   jax.ShapeDtypeStruct((ROWS, 128), _i32)],
    )(e2, s2, skey2, strat2, tsel, escal)


_MESH = plsc.VectorSubcoreMesh(core_axis_name="c", subcore_axis_name="s",
                               num_cores=2, num_subcores=16)


def _zero_ref(ref, nwords):
    z = jnp.zeros((16,), _i32)
    assert nwords % 128 == 0

    def body(j, carry):
        for u in range(8):
            ref[pl.ds(j * 128 + u * 16, 16)] = z
        return carry

    lax.fori_loop(0, nwords // 128, body, 0)


def _fold_lanes(hist, fold, nbins):
    iota = lax.broadcasted_iota(_i32, (16,), 0)
    assert nbins % 32 == 0

    def body(j, carry):
        for u in range(2):
            bins = iota + (j * 32 + u * 16)
            acc = jnp.zeros((16,), _i32)
            for L in range(16):
                acc = acc + plsc.load_gather(hist, [bins * 16 + L])
            fold[pl.ds(j * 32 + u * 16, 16)] = acc
        return carry

    lax.fori_loop(0, nbins // 32, body, 0)


def _sc_wid():
    return lax.axis_index("s") * 2 + lax.axis_index("c")


def _sc_sweep(kbuf_list, nv, per_vec):
    U = 8
    nu = nv // U

    def body(i, carry):
        for u in range(U):
            j = i * U + u
            vecs = [b[pl.ds(j * 16, 16)] for b in kbuf_list]
            per_vec(j, *vecs)
        return carry

    lax.fori_loop(0, nu, body, 0)
    for j in range(nu * U, nv):
        vecs = [b[pl.ds(j * 16, 16)] for b in kbuf_list]
        per_vec(j, *vecs)


def _sc_ehist_body(ekey_ref, part_ref, kbuf, hist, fold, *, level, nbins,
                   outb, histwords, tgt_ref=None, tbuf=None):
    wid = _sc_wid()
    base = wid * (NVT * 16)
    _zero_ref(hist, histwords)
    iota = lax.broadcasted_iota(_i32, (16,), 0)
    ones = jnp.full((16,), 1, _i32)

    if level > 0:
        pltpu.sync_copy(tgt_ref, tbuf)
        t0 = tbuf[pl.ds(0, 16)]
        t1 = tbuf[pl.ds(16, 16)]
        prefs = [t0[t] for t in range(16)] + [t1[0], t1[1]]
        shift = {1: 20, 2: 12, 3: 4}[level]
        submask = np.uint32(0xF if level == 3 else 0xFF)
        subshift = {1: 12, 2: 4, 3: 0}[level]
        nsub = 16 if level == 3 else 256

    off = 0
    for nv in BLOCKS:
        pltpu.sync_copy(ekey_ref.at[pl.ds(base + off * 16, nv * 16)],
                        kbuf.at[pl.ds(0, nv * 16)])

        if level == 0:
            def per_vec(i, kv):
                bin12 = (kv >> np.uint32(20)).astype(_i32)
                plsc.addupdate_scatter(hist, [bin12 * 16 + iota], ones)
        else:
            def per_vec(i, kv):
                v = (kv >> np.uint32(shift)).astype(_i32)
                pos = jnp.zeros((16,), _i32)
                for t in range(18):
                    pos = pos + (prefs[t] <= v).astype(_i32)
                slot = jnp.maximum(pos - 1, 0)
                pref = plsc.load_gather(tbuf, [slot])
                match = (pos > 0) & (pref == v)
                sub = ((kv >> np.uint32(subshift)) & submask).astype(_i32)
                idx = (slot * nsub + sub) * 16 + iota
                plsc.addupdate_scatter(hist, [idx], ones, mask=match)

        _sc_sweep([kbuf], nv, per_vec)
        off += nv

    _zero_ref(fold, outb)
    _fold_lanes(hist, fold, nbins)
    pltpu.sync_copy(fold, part_ref.at[wid])


def _make_sc_ehist(level):
    nbins = {0: 4096, 1: 4608, 2: 4608, 3: 288}[level]
    outb = {0: 4096, 1: 4608, 2: 4608, 3: 384}[level]
    histwords = {0: 65536, 1: 73728, 2: 73728, 3: 6144}[level]
    scratch = [pltpu.VMEM((8192,), _u32),
               pltpu.VMEM((histwords,), _i32),
               pltpu.VMEM((outb,), _i32)]
    if level > 0:
        scratch += [pltpu.VMEM((128,), _i32)]

    def body(*args):
        if level == 0:
            ekey_ref, part_ref, kbuf, hist, fold = args
            _sc_ehist_body(ekey_ref, part_ref, kbuf, hist, fold, level=0,
                           nbins=nbins, outb=outb, histwords=histwords)
        else:
            ekey_ref, tgt_ref, part_ref, kbuf, hist, fold, tbuf = args
            _sc_ehist_body(ekey_ref, part_ref, kbuf, hist, fold, level=level,
                           nbins=nbins, outb=outb, histwords=histwords,
                           tgt_ref=tgt_ref, tbuf=tbuf)

    return pl.kernel(
        body,
        out_type=jax.ShapeDtypeStruct((NT, outb), _i32),
        mesh=_MESH,
        compiler_params=pltpu.CompilerParams(needs_layout_passes=False),
        scratch_types=scratch,
    )


def _sc_shist_body(skey_ref, strat_ref, part_ref, kbuf, sbuf, hist, fold,
                   *, level, tgt_ref=None, tbuf=None):
    wid = _sc_wid()
    base = wid * (NVT * 16)
    _zero_ref(hist, 45056)
    iota = lax.broadcasted_iota(_i32, (16,), 0)
    ones = jnp.full((16,), 1, _i32)

    if level > 0:
        pltpu.sync_copy(tgt_ref, tbuf)
        shift = {1: 24, 2: 16, 3: 8}[level]
        subshift = {1: 16, 2: 8, 3: 0}[level]

    off = 0
    for nv in BLOCKS:
        pltpu.sync_copy(skey_ref.at[pl.ds(base + off * 16, nv * 16)],
                        kbuf.at[pl.ds(0, nv * 16)])
        pltpu.sync_copy(strat_ref.at[pl.ds(base + off * 16, nv * 16)],
                        sbuf.at[pl.ds(0, nv * 16)])

        if level == 0:
            def per_vec(i, kv, st):
                sub = (kv >> np.uint32(24)).astype(_i32)
                plsc.addupdate_scatter(hist, [(st * 256 + sub) * 16 + iota],
                                       ones)
        else:
            def per_vec(i, kv, st):
                pref = plsc.load_gather(tbuf, [st])
                match = (kv >> np.uint32(shift)).astype(_i32) == pref
                sub = ((kv >> np.uint32(subshift)) & np.uint32(0xFF)) \
                    .astype(_i32)
                idx = (st * 256 + sub) * 16 + iota
                plsc.addupdate_scatter(hist, [idx], ones, mask=match)

        _sc_sweep([kbuf, sbuf], nv, per_vec)
        off += nv

    _fold_lanes(hist, fold, 2560)
    pltpu.sync_copy(fold, part_ref.at[wid])


def _make_sc_shist(level):
    scratch = [pltpu.VMEM((8192,), _u32),
               pltpu.VMEM((8192,), _i32),
               pltpu.VMEM((45056,), _i32),
               pltpu.VMEM((2560,), _i32)]
    if level > 0:
        scratch += [pltpu.VMEM((128,), _i32)]

    def body(*args):
        if level == 0:
            skey_ref, strat_ref, part_ref, kbuf, sbuf, hist, fold = args
            _sc_shist_body(skey_ref, strat_ref, part_ref, kbuf, sbuf, hist,
                           fold, level=0)
        else:
            (skey_ref, strat_ref, tgt_ref, part_ref, kbuf, sbuf, hist, fold,
             tbuf) = args
            _sc_shist_body(skey_ref, strat_ref, part_ref, kbuf, sbuf, hist,
                           fold, level=level, tgt_ref=tgt_ref, tbuf=tbuf)

    return pl.kernel(
        body,
        out_type=jax.ShapeDtypeStruct((NT, 2560), _i32),
        mesh=_MESH,
        compiler_params=pltpu.CompilerParams(needs_layout_passes=False),
        scratch_types=scratch,
    )


_SC_EHIST = {lvl: _make_sc_ehist(lvl) for lvl in range(4)}
_SC_SHIST = {lvl: _make_sc_shist(lvl) for lvl in range(4)}


def kernel(entropy_map, node_scales, budget):
    e2 = jnp.pad(entropy_map, (0, PAD - N),
                 constant_values=np.float32(np.inf)).reshape(ROWS, 128)
    s2 = jnp.pad(node_scales.astype(_i32), (0, PAD - N)).reshape(ROWS, 128)

    ekey2, ppart = _tc_ekey(e2, s2)
    ekey1 = ekey2.reshape(PAD)

    eh0 = _SC_EHIST[0](ekey1).reshape(NT, 32, 128)
    etgt, escal = _tc_me0(eh0, ppart)
    eh1 = _SC_EHIST[1](ekey1, etgt.reshape(128)).reshape(NT, 36, 128)
    etgt = _tc_me12(eh1, etgt, 8)
    eh2 = _SC_EHIST[2](ekey1, etgt.reshape(128)).reshape(NT, 36, 128)
    etgt = _tc_me12(eh2, etgt, 8)
    eh3 = _SC_EHIST[3](ekey1, etgt.reshape(128)).reshape(NT, 3, 128)
    qb = _tc_me3(eh3, etgt)

    skey2, strat2 = _tc_skey(e2, s2, qb, escal)
    skey1 = skey2.reshape(PAD)
    strat1 = strat2.reshape(PAD)

    budget = jnp.asarray(budget, _i32)
    g = jnp.gcd(budget, N)
    bscal = jnp.zeros((1, 128), _i32)
    bscal = bscal.at[0, 0].set(budget // g).at[0, 1].set(N // g)

    sh0 = _SC_SHIST[0](skey1, strat1).reshape(NT, 20, 128)
    stgt = _tc_ms(sh0, bscal, bscal, level=0)
    sh1 = _SC_SHIST[1](skey1, strat1, stgt.reshape(128)).reshape(NT, 20, 128)
    stgt = _tc_ms(sh1, stgt, bscal, level=1)
    sh2 = _SC_SHIST[2](skey1, strat1, stgt.reshape(128)).reshape(NT, 20, 128)
    stgt = _tc_ms(sh2, stgt, bscal, level=2)
    sh3 = _SC_SHIST[3](skey1, strat1, stgt.reshape(128)).reshape(NT, 20, 128)
    tsel = _tc_ms3(sh3, stgt)

    probs2, sel2 = _tc_final(e2, s2, skey2, strat2, tsel, escal)
    probs = probs2.reshape(PAD)[:N]
    mask = sel2.reshape(PAD)[:N].astype(jnp.bool_)
    return probs, mask
